# baseline (speedup 1.0000x reference)
import functools

import jax
import jax.numpy as jnp
from jax import lax
from jax.experimental import pallas as pl
from jax.experimental.pallas import tpu as pltpu
from jax.experimental.pallas import tpu_sc as plsc

NC = 2
NS = 16
NW = NC * NS
CH = 128
HW = 16

_NOTILE = pltpu.CompilerParams(use_tc_tiling_on_sc=False)


def _sc_mesh():
    return plsc.VectorSubcoreMesh(
        core_axis_name="c", subcore_axis_name="s", num_cores=NC, num_subcores=NS
    )


def _make_hist(n_pad, ts):
    rpt = n_pad // NS

    def body(edges_hbm, out_hbm, acc, idx_v, ones_v, stage_v, gsem):
        c = lax.axis_index("c")
        s = lax.axis_index("s")
        g = c * NS + s
        ones16 = jnp.full((16,), 1.0, jnp.float32)
        zeros16 = jnp.zeros((16,), jnp.float32)

        def fill_ones(i, _):
            ones_v[i, :] = ones16
            return 0

        lax.fori_loop(0, CH, fill_ones, 0)

        def fill_zero(i, _):
            stage_v[i, :] = zeros16
            return 0

        lax.fori_loop(0, rpt, fill_zero, 0)

        base = s * rpt
        pltpu.sync_copy(stage_v, acc.at[pl.ds(base, rpt)])
        plsc.subcore_barrier()

        def chunk(j, _):
            pltpu.sync_copy(edges_hbm.at[g, j], idx_v)
            pltpu.sync_copy(ones_v, acc.at[idx_v.at[1]], add=True)
            return 0

        lax.fori_loop(0, ts, chunk, 0)
        plsc.subcore_barrier()

        pltpu.sync_copy(acc.at[pl.ds(base, rpt)], stage_v)
        pltpu.sync_copy(stage_v, out_hbm.at[c, pl.ds(base, rpt)])

    return pl.kernel(
        body,
        out_type=jax.ShapeDtypeStruct((NC, n_pad, HW), jnp.float32),
        mesh=_sc_mesh(),
        compiler_params=_NOTILE,
        scratch_types=[
            pltpu.VMEM_SHARED((n_pad, HW), jnp.float32),
            pltpu.VMEM((2, CH), jnp.int32),
            pltpu.VMEM((CH, HW), jnp.float32),
            pltpu.VMEM((rpt, HW), jnp.float32),
            pltpu.SemaphoreType.DMA,
        ],
    )


def _make_hop(n_pad, t_loop, d):
    rpt = n_pad // NS
    blk = 104
    n_full = rpt // blk
    tail = rpt - n_full * blk
    assert tail % 8 == 0 and t_loop % 4 == 0 and t_loop >= 8

    def body(edges_hbm, y_hbm, out_hbm, acc,
             i0, i1, i2, i3, r0, r1, zst,
             is0, is1, is2, is3, gs0, gs1, ss0, ss1):
        c = lax.axis_index("c")
        s = lax.axis_index("s")
        g = c * NS + s
        idxb = [i0, i1, i2, i3]
        rows = [r0, r1]
        isem = [is0, is1, is2, is3]
        gsem = [gs0, gs1]
        ssem = [ss0, ss1]
        zeros16 = jnp.zeros((16,), jnp.float32)

        def fill_zero(i, _):
            def fill_col(j, _):
                zst[i, pl.ds(j * 16, 16)] = zeros16
                return 0

            lax.fori_loop(0, d // 16, fill_col, 0)
            return 0

        lax.fori_loop(0, blk, fill_zero, 0)

        base = s * rpt

        def zero_blk(k, _):
            pltpu.sync_copy(zst, acc.at[pl.ds(base + k * blk, blk)])
            return 0

        lax.fori_loop(0, n_full, zero_blk, 0)
        if tail:
            pltpu.sync_copy(
                zst.at[pl.ds(0, tail)],
                acc.at[pl.ds(base + n_full * blk, tail)],
            )
        plsc.subcore_barrier()

        def wait_rows(sem_, buf):
            pltpu.make_async_copy(y_hbm.at[pl.ds(0, CH)], buf, sem_).wait()

        def step(j, bb, skip_a):
            b = bb % 2
            bn = (bb + 1) % 2
            if not skip_a:
                wait_rows(ssem[bn], rows[bn])
            pltpu.async_copy(
                edges_hbm.at[g, j + 3], idxb[(bb + 3) % 4], isem[(bb + 3) % 4]
            )
            pltpu.make_async_copy(
                edges_hbm.at[g, 0], idxb[(bb + 1) % 4], isem[(bb + 1) % 4]
            ).wait()
            pltpu.async_copy(
                y_hbm.at[idxb[(bb + 1) % 4].at[0]], rows[bn], gsem[bn]
            )
            wait_rows(gsem[b], rows[b])
            pltpu.async_copy(rows[b], acc.at[idxb[bb].at[1]], ssem[b], add=True)

        pltpu.async_copy(edges_hbm.at[g, 0], idxb[0], isem[0])
        pltpu.async_copy(edges_hbm.at[g, 1], idxb[1], isem[1])
        pltpu.async_copy(edges_hbm.at[g, 2], idxb[2], isem[2])
        pltpu.make_async_copy(edges_hbm.at[g, 0], idxb[0], isem[0]).wait()
        pltpu.async_copy(y_hbm.at[idxb[0].at[0]], rows[0], gsem[0])

        for bb in range(4):
            step(bb, bb, skip_a=(bb == 0))

        def group(gi, _):
            j0 = gi * 4
            for bb in range(4):
                step(j0 + bb, bb, skip_a=False)
            return 0

        lax.fori_loop(1, t_loop // 4, group, 0)

        wait_rows(ssem[(t_loop - 1) % 2], rows[(t_loop - 1) % 2])
        wait_rows(gsem[t_loop % 2], rows[t_loop % 2])
        pltpu.make_async_copy(
            edges_hbm.at[g, 0], idxb[(t_loop + 1) % 4], isem[(t_loop + 1) % 4]
        ).wait()
        pltpu.make_async_copy(
            edges_hbm.at[g, 0], idxb[(t_loop + 2) % 4], isem[(t_loop + 2) % 4]
        ).wait()
        plsc.subcore_barrier()

        def read_blk(k, _):
            pltpu.sync_copy(acc.at[pl.ds(base + k * blk, blk)], zst)
            pltpu.sync_copy(zst, out_hbm.at[c, pl.ds(base + k * blk, blk)])
            return 0

        lax.fori_loop(0, n_full, read_blk, 0)
        if tail:
            pltpu.sync_copy(
                acc.at[pl.ds(base + n_full * blk, tail)],
                zst.at[pl.ds(0, tail)],
            )
            pltpu.sync_copy(
                zst.at[pl.ds(0, tail)],
                out_hbm.at[c, pl.ds(base + n_full * blk, tail)],
            )

    return pl.kernel(
        body,
        out_type=jax.ShapeDtypeStruct((NC, n_pad, d), jnp.float32),
        mesh=_sc_mesh(),
        compiler_params=_NOTILE,
        scratch_types=[
            pltpu.VMEM_SHARED((n_pad, d), jnp.float32),
            pltpu.VMEM((2, CH), jnp.int32),
            pltpu.VMEM((2, CH), jnp.int32),
            pltpu.VMEM((2, CH), jnp.int32),
            pltpu.VMEM((2, CH), jnp.int32),
            pltpu.VMEM((CH, d), jnp.float32),
            pltpu.VMEM((CH, d), jnp.float32),
            pltpu.VMEM((blk, d), jnp.float32),
            pltpu.SemaphoreType.DMA,
            pltpu.SemaphoreType.DMA,
            pltpu.SemaphoreType.DMA,
            pltpu.SemaphoreType.DMA,
            pltpu.SemaphoreType.DMA,
            pltpu.SemaphoreType.DMA,
            pltpu.SemaphoreType.DMA,
            pltpu.SemaphoreType.DMA,
        ],
    )


def _tc_scale0(hist_ref, x_ref, dinv_ref, y1_ref):
    deg = 1.0 + hist_ref[0, :, 0:1] + hist_ref[1, :, 0:1]
    dinv = lax.rsqrt(deg)
    dinv_ref[...] = dinv
    y1_ref[...] = x_ref[...] * dinv


def _tc_scale1(part_ref, y1_ref, dinv_ref, y2_ref):
    dv = dinv_ref[...]
    y2_ref[...] = (part_ref[0] + part_ref[1] + y1_ref[...]) * (dv * dv)


def _tc_final(part_ref, y2_ref, dinv_ref, wt_ref, b_ref, o_ref):
    h2 = (part_ref[0] + part_ref[1] + y2_ref[...]) * dinv_ref[...]
    o_ref[...] = (
        jnp.dot(h2, wt_ref[...], preferred_element_type=jnp.float32) + b_ref[...]
    )


def kernel(x, edge_index, W, b):
    n, d = x.shape
    e = edge_index.shape[1]
    assert n % NS == 0 and d % 16 == 0

    t_loop = -(-e // (NW * CH))
    t_loop = -(-t_loop // 4) * 4
    ts = t_loop + 3
    e_pad = NW * t_loop * CH
    n_pad = -(-(n + 1) // (NS * 8)) * (NS * 8)

    src = edge_index[0]
    dst = edge_index[1]
    pad = e_pad - e
    if pad:
        src = jnp.concatenate([src, jnp.zeros((pad,), jnp.int32)])
        dst = jnp.concatenate([dst, jnp.full((pad,), n, jnp.int32)])
    src_m = jnp.concatenate(
        [src.reshape(NW, t_loop, CH), jnp.zeros((NW, 3, CH), jnp.int32)], axis=1
    )
    dst_m = jnp.concatenate(
        [dst.reshape(NW, t_loop, CH), jnp.full((NW, 3, CH), n, jnp.int32)],
        axis=1,
    )
    edges = jnp.stack([src_m, dst_m], axis=2)

    hist = _make_hist(n_pad, ts)(edges)
    hop = _make_hop(n_pad, t_loop, d)

    dinv, y1 = pl.pallas_call(
        _tc_scale0,
        out_shape=[
            jax.ShapeDtypeStruct((n, 1), jnp.float32),
            jax.ShapeDtypeStruct((n, d), jnp.float32),
        ],
    )(hist[:, :n, :], x)

    p = hop(edges, y1)

    br = 2000 if n % 2000 == 0 else n
    grid = n // br
    y2 = pl.pallas_call(
        _tc_scale1,
        grid=(grid,),
        in_specs=[
            pl.BlockSpec((NC, br, d), lambda i: (0, i, 0)),
            pl.BlockSpec((br, d), lambda i: (i, 0)),
            pl.BlockSpec((br, 1), lambda i: (i, 0)),
        ],
        out_specs=pl.BlockSpec((br, d), lambda i: (i, 0)),
        out_shape=jax.ShapeDtypeStruct((n, d), jnp.float32),
    )(p[:, :n, :], y1, dinv)

    q = hop(edges, y2)

    out = pl.pallas_call(
        _tc_final,
        grid=(grid,),
        in_specs=[
            pl.BlockSpec((NC, br, d), lambda i: (0, i, 0)),
            pl.BlockSpec((br, d), lambda i: (i, 0)),
            pl.BlockSpec((br, 1), lambda i: (i, 0)),
            pl.BlockSpec((d, d), lambda i: (0, 0)),
            pl.BlockSpec((1, d), lambda i: (0, 0)),
        ],
        out_specs=pl.BlockSpec((br, d), lambda i: (i, 0)),
        out_shape=jax.ShapeDtypeStruct((n, d), jnp.float32),
    )(q[:, :n, :], y2, dinv, W.T, b.reshape(1, d))
    return out


# speedup vs baseline: 1.1294x; 1.1294x over previous
import functools

import jax
import jax.numpy as jnp
from jax import lax
from jax.experimental import pallas as pl
from jax.experimental.pallas import tpu as pltpu
from jax.experimental.pallas import tpu_sc as plsc

NC = 2
NS = 16
NW = NC * NS
CH = 128
HW = 16

_NOTILE = pltpu.CompilerParams(use_tc_tiling_on_sc=False)


def _sc_mesh():
    return plsc.VectorSubcoreMesh(
        core_axis_name="c", subcore_axis_name="s", num_cores=NC, num_subcores=NS
    )


def _make_hist(n_pad, ts):
    rpt = n_pad // NS

    def body(edges_hbm, out_hbm, acc, idx_v, ones_v, stage_v, gsem):
        c = lax.axis_index("c")
        s = lax.axis_index("s")
        g = c * NS + s
        ones16 = jnp.full((16,), 1.0, jnp.float32)
        zeros16 = jnp.zeros((16,), jnp.float32)

        def fill_ones(i, _):
            ones_v[i, :] = ones16
            return 0

        lax.fori_loop(0, CH, fill_ones, 0)

        def fill_zero(i, _):
            stage_v[i, :] = zeros16
            return 0

        lax.fori_loop(0, rpt, fill_zero, 0)

        base = s * rpt
        pltpu.sync_copy(stage_v, acc.at[pl.ds(base, rpt)])
        plsc.subcore_barrier()

        def chunk(j, _):
            pltpu.sync_copy(edges_hbm.at[g, j], idx_v)
            pltpu.sync_copy(ones_v, acc.at[idx_v.at[1]], add=True)
            return 0

        lax.fori_loop(0, ts, chunk, 0)
        plsc.subcore_barrier()

        pltpu.sync_copy(acc.at[pl.ds(base, rpt)], stage_v)
        pltpu.sync_copy(stage_v, out_hbm.at[c, pl.ds(base, rpt)])

    return pl.kernel(
        body,
        out_type=jax.ShapeDtypeStruct((NC, n_pad, HW), jnp.float32),
        mesh=_sc_mesh(),
        compiler_params=_NOTILE,
        scratch_types=[
            pltpu.VMEM_SHARED((n_pad, HW), jnp.float32),
            pltpu.VMEM((2, CH), jnp.int32),
            pltpu.VMEM((CH, HW), jnp.float32),
            pltpu.VMEM((rpt, HW), jnp.float32),
            pltpu.SemaphoreType.DMA,
        ],
    )


def _make_hop_serial(n_pad, t_loop, d):
    rpt = n_pad // NS
    blk = 104
    n_full = rpt // blk
    tail = rpt - n_full * blk
    assert tail % 8 == 0

    def body(edges_hbm, y_hbm, out_hbm, acc, idx_v, rows_v, zst, gsem):
        c = lax.axis_index("c")
        s = lax.axis_index("s")
        g = c * NS + s
        zeros16 = jnp.zeros((16,), jnp.float32)

        def fill_zero(i, _):
            def fill_col(j, _):
                zst[i, pl.ds(j * 16, 16)] = zeros16
                return 0

            lax.fori_loop(0, d // 16, fill_col, 0)
            return 0

        lax.fori_loop(0, blk, fill_zero, 0)

        base = s * rpt

        def zero_blk(k, _):
            pltpu.sync_copy(zst, acc.at[pl.ds(base + k * blk, blk)])
            return 0

        lax.fori_loop(0, n_full, zero_blk, 0)
        if tail:
            pltpu.sync_copy(
                zst.at[pl.ds(0, tail)],
                acc.at[pl.ds(base + n_full * blk, tail)],
            )
        plsc.subcore_barrier()

        def chunk(j, _):
            pltpu.sync_copy(edges_hbm.at[g, j], idx_v)
            pltpu.async_copy(y_hbm.at[idx_v.at[0]], rows_v, gsem).wait()
            pltpu.sync_copy(rows_v, acc.at[idx_v.at[1]], add=True)
            return 0

        lax.fori_loop(0, t_loop, chunk, 0)
        plsc.subcore_barrier()

        def read_blk(k, _):
            pltpu.sync_copy(acc.at[pl.ds(base + k * blk, blk)], zst)
            pltpu.sync_copy(zst, out_hbm.at[c, pl.ds(base + k * blk, blk)])
            return 0

        lax.fori_loop(0, n_full, read_blk, 0)
        if tail:
            pltpu.sync_copy(
                acc.at[pl.ds(base + n_full * blk, tail)],
                zst.at[pl.ds(0, tail)],
            )
            pltpu.sync_copy(
                zst.at[pl.ds(0, tail)],
                out_hbm.at[c, pl.ds(base + n_full * blk, tail)],
            )

    return pl.kernel(
        body,
        out_type=jax.ShapeDtypeStruct((NC, n_pad, d), jnp.float32),
        mesh=_sc_mesh(),
        compiler_params=_NOTILE,
        scratch_types=[
            pltpu.VMEM_SHARED((n_pad, d), jnp.float32),
            pltpu.VMEM((2, CH), jnp.int32),
            pltpu.VMEM((CH, d), jnp.float32),
            pltpu.VMEM((blk, d), jnp.float32),
            pltpu.SemaphoreType.DMA,
        ],
    )


def _make_hop(n_pad, t_loop, d):
    rpt = n_pad // NS
    blk = 104
    n_full = rpt // blk
    tail = rpt - n_full * blk
    assert tail % 8 == 0 and t_loop % 4 == 0 and t_loop >= 8

    def body(edges_hbm, y_hbm, out_hbm, acc,
             i0, i1, i2, i3, r0, r1, zst,
             is0, is1, is2, is3, gs0, gs1, ss0, ss1):
        c = lax.axis_index("c")
        s = lax.axis_index("s")
        g = c * NS + s
        idxb = [i0, i1, i2, i3]
        rows = [r0, r1]
        isem = [is0, is1, is2, is3]
        gsem = [gs0, gs1]
        ssem = [ss0, ss1]
        zeros16 = jnp.zeros((16,), jnp.float32)

        def fill_zero(i, _):
            def fill_col(j, _):
                zst[i, pl.ds(j * 16, 16)] = zeros16
                return 0

            lax.fori_loop(0, d // 16, fill_col, 0)
            return 0

        lax.fori_loop(0, blk, fill_zero, 0)

        base = s * rpt

        def zero_blk(k, _):
            pltpu.sync_copy(zst, acc.at[pl.ds(base + k * blk, blk)])
            return 0

        lax.fori_loop(0, n_full, zero_blk, 0)
        if tail:
            pltpu.sync_copy(
                zst.at[pl.ds(0, tail)],
                acc.at[pl.ds(base + n_full * blk, tail)],
            )
        plsc.subcore_barrier()

        def wait_rows(sem_, buf):
            pltpu.make_async_copy(y_hbm.at[pl.ds(0, CH)], buf, sem_).wait()

        def step(j, bb, skip_a):
            b = bb % 2
            bn = (bb + 1) % 2
            if not skip_a:
                wait_rows(ssem[bn], rows[bn])
            pltpu.async_copy(
                edges_hbm.at[g, j + 3], idxb[(bb + 3) % 4], isem[(bb + 3) % 4]
            )
            pltpu.make_async_copy(
                edges_hbm.at[g, 0], idxb[(bb + 1) % 4], isem[(bb + 1) % 4]
            ).wait()
            pltpu.async_copy(
                y_hbm.at[idxb[(bb + 1) % 4].at[0]], rows[bn], gsem[bn]
            )
            wait_rows(gsem[b], rows[b])
            pltpu.async_copy(rows[b], acc.at[idxb[bb].at[1]], ssem[b], add=True)

        pltpu.async_copy(edges_hbm.at[g, 0], idxb[0], isem[0])
        pltpu.async_copy(edges_hbm.at[g, 1], idxb[1], isem[1])
        pltpu.async_copy(edges_hbm.at[g, 2], idxb[2], isem[2])
        pltpu.make_async_copy(edges_hbm.at[g, 0], idxb[0], isem[0]).wait()
        pltpu.async_copy(y_hbm.at[idxb[0].at[0]], rows[0], gsem[0])

        for bb in range(4):
            step(bb, bb, skip_a=(bb == 0))

        def group(gi, _):
            j0 = gi * 4
            for bb in range(4):
                step(j0 + bb, bb, skip_a=False)
            return 0

        lax.fori_loop(1, t_loop // 4, group, 0)

        wait_rows(ssem[(t_loop - 1) % 2], rows[(t_loop - 1) % 2])
        wait_rows(gsem[t_loop % 2], rows[t_loop % 2])
        pltpu.make_async_copy(
            edges_hbm.at[g, 0], idxb[(t_loop + 1) % 4], isem[(t_loop + 1) % 4]
        ).wait()
        pltpu.make_async_copy(
            edges_hbm.at[g, 0], idxb[(t_loop + 2) % 4], isem[(t_loop + 2) % 4]
        ).wait()
        plsc.subcore_barrier()

        def read_blk(k, _):
            pltpu.sync_copy(acc.at[pl.ds(base + k * blk, blk)], zst)
            pltpu.sync_copy(zst, out_hbm.at[c, pl.ds(base + k * blk, blk)])
            return 0

        lax.fori_loop(0, n_full, read_blk, 0)
        if tail:
            pltpu.sync_copy(
                acc.at[pl.ds(base + n_full * blk, tail)],
                zst.at[pl.ds(0, tail)],
            )
            pltpu.sync_copy(
                zst.at[pl.ds(0, tail)],
                out_hbm.at[c, pl.ds(base + n_full * blk, tail)],
            )

    return pl.kernel(
        body,
        out_type=jax.ShapeDtypeStruct((NC, n_pad, d), jnp.float32),
        mesh=_sc_mesh(),
        compiler_params=_NOTILE,
        scratch_types=[
            pltpu.VMEM_SHARED((n_pad, d), jnp.float32),
            pltpu.VMEM((2, CH), jnp.int32),
            pltpu.VMEM((2, CH), jnp.int32),
            pltpu.VMEM((2, CH), jnp.int32),
            pltpu.VMEM((2, CH), jnp.int32),
            pltpu.VMEM((CH, d), jnp.float32),
            pltpu.VMEM((CH, d), jnp.float32),
            pltpu.VMEM((blk, d), jnp.float32),
            pltpu.SemaphoreType.DMA,
            pltpu.SemaphoreType.DMA,
            pltpu.SemaphoreType.DMA,
            pltpu.SemaphoreType.DMA,
            pltpu.SemaphoreType.DMA,
            pltpu.SemaphoreType.DMA,
            pltpu.SemaphoreType.DMA,
            pltpu.SemaphoreType.DMA,
        ],
    )


def _tc_scale0(hist_ref, x_ref, dinv_ref, y1_ref):
    deg = 1.0 + hist_ref[0, :, 0:1] + hist_ref[1, :, 0:1]
    dinv = lax.rsqrt(deg)
    dinv_ref[...] = dinv
    y1_ref[...] = x_ref[...] * dinv


def _tc_scale1(part_ref, y1_ref, dinv_ref, y2_ref):
    dv = dinv_ref[...]
    y2_ref[...] = (part_ref[0] + part_ref[1] + y1_ref[...]) * (dv * dv)


def _tc_final(part_ref, y2_ref, dinv_ref, wt_ref, b_ref, o_ref):
    h2 = (part_ref[0] + part_ref[1] + y2_ref[...]) * dinv_ref[...]
    o_ref[...] = (
        jnp.dot(h2, wt_ref[...], preferred_element_type=jnp.float32) + b_ref[...]
    )


def kernel(x, edge_index, W, b):
    n, d = x.shape
    e = edge_index.shape[1]
    assert n % NS == 0 and d % 16 == 0

    t_loop = -(-e // (NW * CH))
    t_loop = -(-t_loop // 4) * 4
    ts = t_loop + 3
    e_pad = NW * t_loop * CH
    n_pad = -(-(n + 1) // (NS * 8)) * (NS * 8)

    src = edge_index[0]
    dst = edge_index[1]
    pad = e_pad - e
    if pad:
        src = jnp.concatenate([src, jnp.zeros((pad,), jnp.int32)])
        dst = jnp.concatenate([dst, jnp.full((pad,), n, jnp.int32)])
    src_m = jnp.concatenate(
        [src.reshape(NW, t_loop, CH), jnp.zeros((NW, 3, CH), jnp.int32)], axis=1
    )
    dst_m = jnp.concatenate(
        [dst.reshape(NW, t_loop, CH), jnp.full((NW, 3, CH), n, jnp.int32)],
        axis=1,
    )
    edges = jnp.stack([src_m, dst_m], axis=2)

    hist = _make_hist(n_pad, ts)(edges)
    hop = _make_hop_serial(n_pad, t_loop, d)

    dinv, y1 = pl.pallas_call(
        _tc_scale0,
        out_shape=[
            jax.ShapeDtypeStruct((n, 1), jnp.float32),
            jax.ShapeDtypeStruct((n, d), jnp.float32),
        ],
    )(hist[:, :n, :], x)

    p = hop(edges, y1)

    br = 2000 if n % 2000 == 0 else n
    grid = n // br
    y2 = pl.pallas_call(
        _tc_scale1,
        grid=(grid,),
        in_specs=[
            pl.BlockSpec((NC, br, d), lambda i: (0, i, 0)),
            pl.BlockSpec((br, d), lambda i: (i, 0)),
            pl.BlockSpec((br, 1), lambda i: (i, 0)),
        ],
        out_specs=pl.BlockSpec((br, d), lambda i: (i, 0)),
        out_shape=jax.ShapeDtypeStruct((n, d), jnp.float32),
    )(p[:, :n, :], y1, dinv)

    q = hop(edges, y2)

    out = pl.pallas_call(
        _tc_final,
        grid=(grid,),
        in_specs=[
            pl.BlockSpec((NC, br, d), lambda i: (0, i, 0)),
            pl.BlockSpec((br, d), lambda i: (i, 0)),
            pl.BlockSpec((br, 1), lambda i: (i, 0)),
            pl.BlockSpec((d, d), lambda i: (0, 0)),
            pl.BlockSpec((1, d), lambda i: (0, 0)),
        ],
        out_specs=pl.BlockSpec((br, d), lambda i: (i, 0)),
        out_shape=jax.ShapeDtypeStruct((n, d), jnp.float32),
    )(q[:, :n, :], y2, dinv, W.T, b.reshape(1, d))
    return out


# speedup vs baseline: 1.3104x; 1.1603x over previous
import functools

import jax
import jax.numpy as jnp
from jax import lax
from jax.experimental import pallas as pl
from jax.experimental.pallas import tpu as pltpu
from jax.experimental.pallas import tpu_sc as plsc

NC = 2
NS = 16
NW = NC * NS
CH = 128
HW = 16

_NOTILE = pltpu.CompilerParams(use_tc_tiling_on_sc=False)


def _sc_mesh():
    return plsc.VectorSubcoreMesh(
        core_axis_name="c", subcore_axis_name="s", num_cores=NC, num_subcores=NS
    )


def _make_hist(n_pad, ts):
    rpt = n_pad // NS

    def body(edges_hbm, out_hbm, acc, idx_v, ones_v, stage_v, gsem):
        c = lax.axis_index("c")
        s = lax.axis_index("s")
        g = c * NS + s
        ones16 = jnp.full((16,), 1.0, jnp.float32)
        zeros16 = jnp.zeros((16,), jnp.float32)

        def fill_ones(i, _):
            ones_v[i, :] = ones16
            return 0

        lax.fori_loop(0, CH, fill_ones, 0)

        def fill_zero(i, _):
            stage_v[i, :] = zeros16
            return 0

        lax.fori_loop(0, rpt, fill_zero, 0)

        base = s * rpt
        pltpu.sync_copy(stage_v, acc.at[pl.ds(base, rpt)])
        plsc.subcore_barrier()

        def chunk(j, _):
            pltpu.sync_copy(edges_hbm.at[g, j], idx_v)
            pltpu.sync_copy(ones_v, acc.at[idx_v], add=True)
            return 0

        lax.fori_loop(0, ts, chunk, 0)
        plsc.subcore_barrier()

        pltpu.sync_copy(acc.at[pl.ds(base, rpt)], stage_v)
        pltpu.sync_copy(stage_v, out_hbm.at[c, pl.ds(base, rpt)])

    return pl.kernel(
        body,
        out_type=jax.ShapeDtypeStruct((NC, n_pad, HW), jnp.float32),
        mesh=_sc_mesh(),
        compiler_params=_NOTILE,
        scratch_types=[
            pltpu.VMEM_SHARED((n_pad, HW), jnp.float32),
            pltpu.VMEM((CH,), jnp.int32),
            pltpu.VMEM((CH, HW), jnp.float32),
            pltpu.VMEM((rpt, HW), jnp.float32),
            pltpu.SemaphoreType.DMA,
        ],
    )


def _make_hop_serial(n_pad, t_loop, d):
    rpt = n_pad // NS
    blk = 104
    n_full = rpt // blk
    tail = rpt - n_full * blk
    assert tail % 8 == 0

    def body(edges_hbm, y_hbm, out_hbm, acc, idx_v, rows_v, zst, gsem):
        c = lax.axis_index("c")
        s = lax.axis_index("s")
        g = c * NS + s
        zeros16 = jnp.zeros((16,), jnp.float32)

        def fill_zero(i, _):
            def fill_col(j, _):
                zst[i, pl.ds(j * 16, 16)] = zeros16
                return 0

            lax.fori_loop(0, d // 16, fill_col, 0)
            return 0

        lax.fori_loop(0, blk, fill_zero, 0)

        base = s * rpt

        def zero_blk(k, _):
            pltpu.sync_copy(zst, acc.at[pl.ds(base + k * blk, blk)])
            return 0

        lax.fori_loop(0, n_full, zero_blk, 0)
        if tail:
            pltpu.sync_copy(
                zst.at[pl.ds(0, tail)],
                acc.at[pl.ds(base + n_full * blk, tail)],
            )
        plsc.subcore_barrier()

        def chunk(j, _):
            pltpu.sync_copy(edges_hbm.at[g, j], idx_v)
            pltpu.async_copy(y_hbm.at[idx_v.at[0]], rows_v, gsem).wait()
            pltpu.sync_copy(rows_v, acc.at[idx_v.at[1]], add=True)
            return 0

        lax.fori_loop(0, t_loop, chunk, 0)
        plsc.subcore_barrier()

        def read_blk(k, _):
            pltpu.sync_copy(acc.at[pl.ds(base + k * blk, blk)], zst)
            pltpu.sync_copy(zst, out_hbm.at[c, pl.ds(base + k * blk, blk)])
            return 0

        lax.fori_loop(0, n_full, read_blk, 0)
        if tail:
            pltpu.sync_copy(
                acc.at[pl.ds(base + n_full * blk, tail)],
                zst.at[pl.ds(0, tail)],
            )
            pltpu.sync_copy(
                zst.at[pl.ds(0, tail)],
                out_hbm.at[c, pl.ds(base + n_full * blk, tail)],
            )

    return pl.kernel(
        body,
        out_type=jax.ShapeDtypeStruct((NC, n_pad, d), jnp.float32),
        mesh=_sc_mesh(),
        compiler_params=_NOTILE,
        scratch_types=[
            pltpu.VMEM_SHARED((n_pad, d), jnp.float32),
            pltpu.VMEM((2, CH), jnp.int32),
            pltpu.VMEM((CH, d), jnp.float32),
            pltpu.VMEM((blk, d), jnp.float32),
            pltpu.SemaphoreType.DMA,
        ],
    )


def _make_hop_v3(n_pad, n_groups, d):
    rpt = n_pad // NS
    blk = 80
    n_full = rpt // blk
    tail = rpt - n_full * blk
    assert tail % 8 == 0 and n_groups % 2 == 0 and n_groups >= 4
    t_last = 4 * n_groups - 1

    def body(edges_hbm, y_hbm, out_hbm, acc, ib0, ib1, r0, r1, zst,
             is0, is1, gs0, gs1, ss0, ss1):
        c = lax.axis_index("c")
        s = lax.axis_index("s")
        g = c * NS + s
        idxb = [ib0, ib1]
        rows = [r0, r1]
        isem = [is0, is1]
        gsem = [gs0, gs1]
        ssem = [ss0, ss1]
        zeros16 = jnp.zeros((16,), jnp.float32)

        def fill_zero(i, _):
            def fill_col(j, _):
                zst[i, pl.ds(j * 16, 16)] = zeros16
                return 0

            lax.fori_loop(0, d // 16, fill_col, 0)
            return 0

        lax.fori_loop(0, blk, fill_zero, 0)

        base = s * rpt

        def zero_blk(k, _):
            pltpu.sync_copy(zst, acc.at[pl.ds(base + k * blk, blk)])
            return 0

        lax.fori_loop(0, n_full, zero_blk, 0)
        if tail:
            pltpu.sync_copy(
                zst.at[pl.ds(0, tail)],
                acc.at[pl.ds(base + n_full * blk, tail)],
            )
        plsc.subcore_barrier()

        def wait_rows(sem_, buf):
            pltpu.make_async_copy(y_hbm.at[pl.ds(0, CH)], buf, sem_).wait()

        def wait_idx(p):
            pltpu.make_async_copy(edges_hbm.at[g, 0], idxb[p], isem[p]).wait()

        def start_idx(q, p):
            pltpu.async_copy(edges_hbm.at[g, q], idxb[p], isem[p])

        def start_gather(p, kk, b):
            pltpu.async_copy(
                y_hbm.at[idxb[p].at[2 * kk]], rows[b], gsem[b]
            )

        def start_scatter(p, kk, b):
            pltpu.async_copy(
                rows[b], acc.at[idxb[p].at[2 * kk + 1]], ssem[b], add=True
            )

        start_idx(0, 0)
        wait_idx(0)
        start_gather(0, 0, 0)
        start_idx(1, 1)
        start_gather(0, 1, 1)
        wait_rows(gsem[0], rows[0])
        start_scatter(0, 0, 0)
        for kk in (2, 3):
            b = kk % 2
            wait_rows(ssem[b], rows[b])
            start_gather(0, kk, b)
            wait_rows(gsem[1 - b], rows[1 - b])
            start_scatter(0, kk - 1, 1 - b)
        wait_idx(1)
        wait_rows(ssem[0], rows[0])
        start_gather(1, 0, 0)
        wait_rows(gsem[1], rows[1])
        start_scatter(0, 3, 1)
        wait_rows(ssem[1], rows[1])
        start_idx(2, 0)
        start_gather(1, 1, 1)
        wait_rows(gsem[0], rows[0])
        start_scatter(1, 0, 0)
        for kk in (2, 3):
            b = kk % 2
            wait_rows(ssem[b], rows[b])
            start_gather(1, kk, b)
            wait_rows(gsem[1 - b], rows[1 - b])
            start_scatter(1, kk - 1, 1 - b)

        def pair(i, _):
            qe = 2 + 2 * i
            for po in ((0, 1, qe), (1, 0, qe + 1)):
                p, pprev, q = po
                wait_idx(p)
                wait_rows(ssem[0], rows[0])
                start_gather(p, 0, 0)
                wait_rows(gsem[1], rows[1])
                start_scatter(pprev, 3, 1)
                wait_rows(ssem[1], rows[1])
                start_idx(q + 1, pprev)
                start_gather(p, 1, 1)
                wait_rows(gsem[0], rows[0])
                start_scatter(p, 0, 0)
                for kk in (2, 3):
                    b = kk % 2
                    wait_rows(ssem[b], rows[b])
                    start_gather(p, kk, b)
                    wait_rows(gsem[1 - b], rows[1 - b])
                    start_scatter(p, kk - 1, 1 - b)
            return 0

        lax.fori_loop(0, (n_groups - 2) // 2, pair, 0)

        p_last = (n_groups - 1) % 2
        wait_rows(gsem[t_last % 2], rows[t_last % 2])
        start_scatter(p_last, 3, t_last % 2)
        wait_rows(ssem[0], rows[0])
        wait_rows(ssem[1], rows[1])
        wait_idx(n_groups % 2)
        plsc.subcore_barrier()

        def read_blk(k, _):
            pltpu.sync_copy(acc.at[pl.ds(base + k * blk, blk)], zst)
            pltpu.sync_copy(zst, out_hbm.at[c, pl.ds(base + k * blk, blk)])
            return 0

        lax.fori_loop(0, n_full, read_blk, 0)
        if tail:
            pltpu.sync_copy(
                acc.at[pl.ds(base + n_full * blk, tail)],
                zst.at[pl.ds(0, tail)],
            )
            pltpu.sync_copy(
                zst.at[pl.ds(0, tail)],
                out_hbm.at[c, pl.ds(base + n_full * blk, tail)],
            )

    return pl.kernel(
        body,
        out_type=jax.ShapeDtypeStruct((NC, n_pad, d), jnp.float32),
        mesh=_sc_mesh(),
        scratch_types=[
            pltpu.VMEM_SHARED((n_pad, d), jnp.float32),
            pltpu.VMEM((8, CH), jnp.int32),
            pltpu.VMEM((8, CH), jnp.int32),
            pltpu.VMEM((CH, d), jnp.float32),
            pltpu.VMEM((CH, d), jnp.float32),
            pltpu.VMEM((blk, d), jnp.float32),
            pltpu.SemaphoreType.DMA,
            pltpu.SemaphoreType.DMA,
            pltpu.SemaphoreType.DMA,
            pltpu.SemaphoreType.DMA,
            pltpu.SemaphoreType.DMA,
            pltpu.SemaphoreType.DMA,
        ],
    )


def _make_hop(n_pad, t_loop, d):
    rpt = n_pad // NS
    blk = 104
    n_full = rpt // blk
    tail = rpt - n_full * blk
    assert tail % 8 == 0 and t_loop % 4 == 0 and t_loop >= 8

    def body(edges_hbm, y_hbm, out_hbm, acc,
             i0, i1, i2, i3, r0, r1, zst,
             is0, is1, is2, is3, gs0, gs1, ss0, ss1):
        c = lax.axis_index("c")
        s = lax.axis_index("s")
        g = c * NS + s
        idxb = [i0, i1, i2, i3]
        rows = [r0, r1]
        isem = [is0, is1, is2, is3]
        gsem = [gs0, gs1]
        ssem = [ss0, ss1]
        zeros16 = jnp.zeros((16,), jnp.float32)

        def fill_zero(i, _):
            def fill_col(j, _):
                zst[i, pl.ds(j * 16, 16)] = zeros16
                return 0

            lax.fori_loop(0, d // 16, fill_col, 0)
            return 0

        lax.fori_loop(0, blk, fill_zero, 0)

        base = s * rpt

        def zero_blk(k, _):
            pltpu.sync_copy(zst, acc.at[pl.ds(base + k * blk, blk)])
            return 0

        lax.fori_loop(0, n_full, zero_blk, 0)
        if tail:
            pltpu.sync_copy(
                zst.at[pl.ds(0, tail)],
                acc.at[pl.ds(base + n_full * blk, tail)],
            )
        plsc.subcore_barrier()

        def wait_rows(sem_, buf):
            pltpu.make_async_copy(y_hbm.at[pl.ds(0, CH)], buf, sem_).wait()

        def step(j, bb, skip_a):
            b = bb % 2
            bn = (bb + 1) % 2
            if not skip_a:
                wait_rows(ssem[bn], rows[bn])
            pltpu.async_copy(
                edges_hbm.at[g, j + 3], idxb[(bb + 3) % 4], isem[(bb + 3) % 4]
            )
            pltpu.make_async_copy(
                edges_hbm.at[g, 0], idxb[(bb + 1) % 4], isem[(bb + 1) % 4]
            ).wait()
            pltpu.async_copy(
                y_hbm.at[idxb[(bb + 1) % 4].at[0]], rows[bn], gsem[bn]
            )
            wait_rows(gsem[b], rows[b])
            pltpu.async_copy(rows[b], acc.at[idxb[bb].at[1]], ssem[b], add=True)

        pltpu.async_copy(edges_hbm.at[g, 0], idxb[0], isem[0])
        pltpu.async_copy(edges_hbm.at[g, 1], idxb[1], isem[1])
        pltpu.async_copy(edges_hbm.at[g, 2], idxb[2], isem[2])
        pltpu.make_async_copy(edges_hbm.at[g, 0], idxb[0], isem[0]).wait()
        pltpu.async_copy(y_hbm.at[idxb[0].at[0]], rows[0], gsem[0])

        for bb in range(4):
            step(bb, bb, skip_a=(bb == 0))

        def group(gi, _):
            j0 = gi * 4
            for bb in range(4):
                step(j0 + bb, bb, skip_a=False)
            return 0

        lax.fori_loop(1, t_loop // 4, group, 0)

        wait_rows(ssem[(t_loop - 1) % 2], rows[(t_loop - 1) % 2])
        wait_rows(gsem[t_loop % 2], rows[t_loop % 2])
        pltpu.make_async_copy(
            edges_hbm.at[g, 0], idxb[(t_loop + 1) % 4], isem[(t_loop + 1) % 4]
        ).wait()
        pltpu.make_async_copy(
            edges_hbm.at[g, 0], idxb[(t_loop + 2) % 4], isem[(t_loop + 2) % 4]
        ).wait()
        plsc.subcore_barrier()

        def read_blk(k, _):
            pltpu.sync_copy(acc.at[pl.ds(base + k * blk, blk)], zst)
            pltpu.sync_copy(zst, out_hbm.at[c, pl.ds(base + k * blk, blk)])
            return 0

        lax.fori_loop(0, n_full, read_blk, 0)
        if tail:
            pltpu.sync_copy(
                acc.at[pl.ds(base + n_full * blk, tail)],
                zst.at[pl.ds(0, tail)],
            )
            pltpu.sync_copy(
                zst.at[pl.ds(0, tail)],
                out_hbm.at[c, pl.ds(base + n_full * blk, tail)],
            )

    return pl.kernel(
        body,
        out_type=jax.ShapeDtypeStruct((NC, n_pad, d), jnp.float32),
        mesh=_sc_mesh(),
        compiler_params=_NOTILE,
        scratch_types=[
            pltpu.VMEM_SHARED((n_pad, d), jnp.float32),
            pltpu.VMEM((2, CH), jnp.int32),
            pltpu.VMEM((2, CH), jnp.int32),
            pltpu.VMEM((2, CH), jnp.int32),
            pltpu.VMEM((2, CH), jnp.int32),
            pltpu.VMEM((CH, d), jnp.float32),
            pltpu.VMEM((CH, d), jnp.float32),
            pltpu.VMEM((blk, d), jnp.float32),
            pltpu.SemaphoreType.DMA,
            pltpu.SemaphoreType.DMA,
            pltpu.SemaphoreType.DMA,
            pltpu.SemaphoreType.DMA,
            pltpu.SemaphoreType.DMA,
            pltpu.SemaphoreType.DMA,
            pltpu.SemaphoreType.DMA,
            pltpu.SemaphoreType.DMA,
        ],
    )


def _tc_scale0(hist_ref, x_ref, dinv_ref, y1_ref):
    deg = 1.0 + hist_ref[0, :, 0:1] + hist_ref[1, :, 0:1]
    dinv = lax.rsqrt(deg)
    dinv_ref[...] = dinv
    y1_ref[...] = x_ref[...] * dinv


def _tc_scale1(part_ref, y1_ref, dinv_ref, y2_ref):
    dv = dinv_ref[...]
    y2_ref[...] = (part_ref[0] + part_ref[1] + y1_ref[...]) * (dv * dv)


def _tc_final(part_ref, y2_ref, dinv_ref, wt_ref, b_ref, o_ref):
    h2 = (part_ref[0] + part_ref[1] + y2_ref[...]) * dinv_ref[...]
    o_ref[...] = (
        jnp.dot(h2, wt_ref[...], preferred_element_type=jnp.float32) + b_ref[...]
    )


def kernel(x, edge_index, W, b):
    n, d = x.shape
    e = edge_index.shape[1]
    assert n % NS == 0 and d % 16 == 0

    t_loop = -(-e // (NW * CH))
    t_loop = -(-t_loop // 8) * 8
    n_groups = t_loop // 4
    e_pad = NW * t_loop * CH
    n_pad = -(-(n + 1) // (NS * 8)) * (NS * 8)

    src = edge_index[0]
    dst = edge_index[1]
    pad = e_pad - e
    if pad:
        src = jnp.concatenate([src, jnp.zeros((pad,), jnp.int32)])
        dst = jnp.concatenate([dst, jnp.full((pad,), n, jnp.int32)])
    dst_m = dst.reshape(NW, t_loop, CH)
    s4 = src.reshape(NW, n_groups, 4, CH)
    d4 = dst.reshape(NW, n_groups, 4, CH)
    inter = jnp.stack([s4, d4], axis=3).reshape(NW, n_groups, 8, CH)
    padg = jnp.stack(
        [
            jnp.zeros((NW, 1, 4, CH), jnp.int32),
            jnp.full((NW, 1, 4, CH), n, jnp.int32),
        ],
        axis=3,
    ).reshape(NW, 1, 8, CH)
    edges8 = jnp.concatenate([inter, padg], axis=1)

    hist = _make_hist(n_pad, t_loop)(dst_m)
    hop = _make_hop_v3(n_pad, n_groups, d)

    dinv, y1 = pl.pallas_call(
        _tc_scale0,
        out_shape=[
            jax.ShapeDtypeStruct((n, 1), jnp.float32),
            jax.ShapeDtypeStruct((n, d), jnp.float32),
        ],
    )(hist[:, :n, :], x)

    p = hop(edges8, y1)

    br = 2000 if n % 2000 == 0 else n
    grid = n // br
    y2 = pl.pallas_call(
        _tc_scale1,
        grid=(grid,),
        in_specs=[
            pl.BlockSpec((NC, br, d), lambda i: (0, i, 0)),
            pl.BlockSpec((br, d), lambda i: (i, 0)),
            pl.BlockSpec((br, 1), lambda i: (i, 0)),
        ],
        out_specs=pl.BlockSpec((br, d), lambda i: (i, 0)),
        out_shape=jax.ShapeDtypeStruct((n, d), jnp.float32),
    )(p[:, :n, :], y1, dinv)

    q = hop(edges8, y2)

    out = pl.pallas_call(
        _tc_final,
        grid=(grid,),
        in_specs=[
            pl.BlockSpec((NC, br, d), lambda i: (0, i, 0)),
            pl.BlockSpec((br, d), lambda i: (i, 0)),
            pl.BlockSpec((br, 1), lambda i: (i, 0)),
            pl.BlockSpec((d, d), lambda i: (0, 0)),
            pl.BlockSpec((1, d), lambda i: (0, 0)),
        ],
        out_specs=pl.BlockSpec((br, d), lambda i: (i, 0)),
        out_shape=jax.ShapeDtypeStruct((n, d), jnp.float32),
    )(q[:, :n, :], y2, dinv, W.T, b.reshape(1, d))
    return out


# speedup vs baseline: 1.3157x; 1.0040x over previous
import functools

import jax
import jax.numpy as jnp
from jax import lax
from jax.experimental import pallas as pl
from jax.experimental.pallas import tpu as pltpu
from jax.experimental.pallas import tpu_sc as plsc

NC = 2
SPLIT_A = 0.2
NS = 16
NW = NC * NS
CH = 128
HW = 16

_NOTILE = pltpu.CompilerParams(use_tc_tiling_on_sc=False)


def _sc_mesh():
    return plsc.VectorSubcoreMesh(
        core_axis_name="c", subcore_axis_name="s", num_cores=NC, num_subcores=NS
    )


def _make_hist(n_pad, ts):
    rpt = n_pad // NS

    def body(edges_hbm, out_hbm, acc, idx_v, ones_v, stage_v, gsem):
        c = lax.axis_index("c")
        s = lax.axis_index("s")
        g = c * NS + s
        ones16 = jnp.full((16,), 1.0, jnp.float32)
        zeros16 = jnp.zeros((16,), jnp.float32)

        def fill_ones(i, _):
            ones_v[i, :] = ones16
            return 0

        lax.fori_loop(0, CH, fill_ones, 0)

        def fill_zero(i, _):
            stage_v[i, :] = zeros16
            return 0

        lax.fori_loop(0, rpt, fill_zero, 0)

        base = s * rpt
        pltpu.sync_copy(stage_v, acc.at[pl.ds(base, rpt)])
        plsc.subcore_barrier()

        def chunk(j, _):
            pltpu.sync_copy(edges_hbm.at[g, j], idx_v)
            pltpu.sync_copy(ones_v, acc.at[idx_v], add=True)
            return 0

        lax.fori_loop(0, ts, chunk, 0)
        plsc.subcore_barrier()

        pltpu.sync_copy(acc.at[pl.ds(base, rpt)], stage_v)
        pltpu.sync_copy(stage_v, out_hbm.at[c, pl.ds(base, rpt)])

    return pl.kernel(
        body,
        out_type=jax.ShapeDtypeStruct((NC, n_pad, HW), jnp.float32),
        mesh=_sc_mesh(),
        compiler_params=_NOTILE,
        scratch_types=[
            pltpu.VMEM_SHARED((n_pad, HW), jnp.float32),
            pltpu.VMEM((CH,), jnp.int32),
            pltpu.VMEM((CH, HW), jnp.float32),
            pltpu.VMEM((rpt, HW), jnp.float32),
            pltpu.SemaphoreType.DMA,
        ],
    )


def _make_hop_serial(n_pad, t_loop, d):
    rpt = n_pad // NS
    blk = 104
    n_full = rpt // blk
    tail = rpt - n_full * blk
    assert tail % 8 == 0

    def body(edges_hbm, y_hbm, out_hbm, acc, idx_v, rows_v, zst, gsem):
        c = lax.axis_index("c")
        s = lax.axis_index("s")
        g = c * NS + s
        zeros16 = jnp.zeros((16,), jnp.float32)

        def fill_zero(i, _):
            def fill_col(j, _):
                zst[i, pl.ds(j * 16, 16)] = zeros16
                return 0

            lax.fori_loop(0, d // 16, fill_col, 0)
            return 0

        lax.fori_loop(0, blk, fill_zero, 0)

        base = s * rpt

        def zero_blk(k, _):
            pltpu.sync_copy(zst, acc.at[pl.ds(base + k * blk, blk)])
            return 0

        lax.fori_loop(0, n_full, zero_blk, 0)
        if tail:
            pltpu.sync_copy(
                zst.at[pl.ds(0, tail)],
                acc.at[pl.ds(base + n_full * blk, tail)],
            )
        plsc.subcore_barrier()

        def chunk(j, _):
            pltpu.sync_copy(edges_hbm.at[g, j], idx_v)
            pltpu.async_copy(y_hbm.at[idx_v.at[0]], rows_v, gsem).wait()
            pltpu.sync_copy(rows_v, acc.at[idx_v.at[1]], add=True)
            return 0

        lax.fori_loop(0, t_loop, chunk, 0)
        plsc.subcore_barrier()

        def read_blk(k, _):
            pltpu.sync_copy(acc.at[pl.ds(base + k * blk, blk)], zst)
            pltpu.sync_copy(zst, out_hbm.at[c, pl.ds(base + k * blk, blk)])
            return 0

        lax.fori_loop(0, n_full, read_blk, 0)
        if tail:
            pltpu.sync_copy(
                acc.at[pl.ds(base + n_full * blk, tail)],
                zst.at[pl.ds(0, tail)],
            )
            pltpu.sync_copy(
                zst.at[pl.ds(0, tail)],
                out_hbm.at[c, pl.ds(base + n_full * blk, tail)],
            )

    return pl.kernel(
        body,
        out_type=jax.ShapeDtypeStruct((NC, n_pad, d), jnp.float32),
        mesh=_sc_mesh(),
        compiler_params=_NOTILE,
        scratch_types=[
            pltpu.VMEM_SHARED((n_pad, d), jnp.float32),
            pltpu.VMEM((2, CH), jnp.int32),
            pltpu.VMEM((CH, d), jnp.float32),
            pltpu.VMEM((blk, d), jnp.float32),
            pltpu.SemaphoreType.DMA,
        ],
    )


def _make_hop_v3(n_pad, ga, gb, d):
    rpt = n_pad // NS
    blk = 80
    n_full = rpt // blk
    tail = rpt - n_full * blk
    assert tail % 8 == 0
    assert ga % 2 == 0 and gb % 2 == 0 and min(ga, gb) >= 4

    def body(edges_hbm, y_hbm, out_hbm, acc, ib0, ib1, r0, r1, zst,
             is0, is1, gs0, gs1, ss0, ss1):
        c = lax.axis_index("c")
        s = lax.axis_index("s")
        ng = jnp.where(c == 0, ga, gb)
        qbase = jnp.where(c == 0, s * ga, NS * ga + s * gb)
        idxb = [ib0, ib1]
        rows = [r0, r1]
        isem = [is0, is1]
        gsem = [gs0, gs1]
        ssem = [ss0, ss1]
        zeros16 = jnp.zeros((16,), jnp.float32)

        def fill_zero(i, _):
            def fill_col(j, _):
                zst[i, pl.ds(j * 16, 16)] = zeros16
                return 0

            lax.fori_loop(0, d // 16, fill_col, 0)
            return 0

        lax.fori_loop(0, blk, fill_zero, 0)

        base = s * rpt

        def zero_blk(k, _):
            pltpu.sync_copy(zst, acc.at[pl.ds(base + k * blk, blk)])
            return 0

        lax.fori_loop(0, n_full, zero_blk, 0)
        if tail:
            pltpu.sync_copy(
                zst.at[pl.ds(0, tail)],
                acc.at[pl.ds(base + n_full * blk, tail)],
            )
        plsc.subcore_barrier()

        def wait_rows(sem_, buf):
            pltpu.make_async_copy(y_hbm.at[pl.ds(0, CH)], buf, sem_).wait()

        def wait_idx(p):
            pltpu.make_async_copy(edges_hbm.at[qbase], idxb[p], isem[p]).wait()

        def start_idx(q, p):
            pltpu.async_copy(edges_hbm.at[qbase + q], idxb[p], isem[p])

        def start_gather(p, kk, b):
            pltpu.async_copy(
                y_hbm.at[idxb[p].at[2 * kk]], rows[b], gsem[b]
            )

        def start_scatter(p, kk, b):
            pltpu.async_copy(
                rows[b], acc.at[idxb[p].at[2 * kk + 1]], ssem[b], add=True
            )

        start_idx(0, 0)
        wait_idx(0)
        start_gather(0, 0, 0)
        start_idx(1, 1)
        start_gather(0, 1, 1)
        wait_rows(gsem[0], rows[0])
        start_scatter(0, 0, 0)
        for kk in (2, 3):
            b = kk % 2
            wait_rows(ssem[b], rows[b])
            start_gather(0, kk, b)
            wait_rows(gsem[1 - b], rows[1 - b])
            start_scatter(0, kk - 1, 1 - b)
        wait_idx(1)
        wait_rows(ssem[0], rows[0])
        start_gather(1, 0, 0)
        wait_rows(gsem[1], rows[1])
        start_scatter(0, 3, 1)
        wait_rows(ssem[1], rows[1])
        start_idx(2, 0)
        start_gather(1, 1, 1)
        wait_rows(gsem[0], rows[0])
        start_scatter(1, 0, 0)
        for kk in (2, 3):
            b = kk % 2
            wait_rows(ssem[b], rows[b])
            start_gather(1, kk, b)
            wait_rows(gsem[1 - b], rows[1 - b])
            start_scatter(1, kk - 1, 1 - b)

        def pair(i, _):
            qe = 2 + 2 * i
            for po in ((0, 1, qe), (1, 0, qe + 1)):
                p, pprev, q = po
                wait_idx(p)
                wait_rows(ssem[0], rows[0])
                start_gather(p, 0, 0)
                wait_rows(gsem[1], rows[1])
                start_scatter(pprev, 3, 1)
                wait_rows(ssem[1], rows[1])
                start_idx(q + 1, pprev)
                start_gather(p, 1, 1)
                wait_rows(gsem[0], rows[0])
                start_scatter(p, 0, 0)
                for kk in (2, 3):
                    b = kk % 2
                    wait_rows(ssem[b], rows[b])
                    start_gather(p, kk, b)
                    wait_rows(gsem[1 - b], rows[1 - b])
                    start_scatter(p, kk - 1, 1 - b)
            return 0

        lax.fori_loop(0, (ng - 2) // 2, pair, 0)

        wait_rows(gsem[1], rows[1])
        start_scatter(1, 3, 1)
        wait_rows(ssem[0], rows[0])
        wait_rows(ssem[1], rows[1])
        wait_idx(0)
        plsc.subcore_barrier()

        def read_blk(k, _):
            pltpu.sync_copy(acc.at[pl.ds(base + k * blk, blk)], zst)
            pltpu.sync_copy(zst, out_hbm.at[c, pl.ds(base + k * blk, blk)])
            return 0

        lax.fori_loop(0, n_full, read_blk, 0)
        if tail:
            pltpu.sync_copy(
                acc.at[pl.ds(base + n_full * blk, tail)],
                zst.at[pl.ds(0, tail)],
            )
            pltpu.sync_copy(
                zst.at[pl.ds(0, tail)],
                out_hbm.at[c, pl.ds(base + n_full * blk, tail)],
            )

    return pl.kernel(
        body,
        out_type=jax.ShapeDtypeStruct((NC, n_pad, d), jnp.float32),
        mesh=_sc_mesh(),
        scratch_types=[
            pltpu.VMEM_SHARED((n_pad, d), jnp.float32),
            pltpu.VMEM((8, CH), jnp.int32),
            pltpu.VMEM((8, CH), jnp.int32),
            pltpu.VMEM((CH, d), jnp.float32),
            pltpu.VMEM((CH, d), jnp.float32),
            pltpu.VMEM((blk, d), jnp.float32),
            pltpu.SemaphoreType.DMA,
            pltpu.SemaphoreType.DMA,
            pltpu.SemaphoreType.DMA,
            pltpu.SemaphoreType.DMA,
            pltpu.SemaphoreType.DMA,
            pltpu.SemaphoreType.DMA,
        ],
    )


def _make_hop(n_pad, t_loop, d):
    rpt = n_pad // NS
    blk = 104
    n_full = rpt // blk
    tail = rpt - n_full * blk
    assert tail % 8 == 0 and t_loop % 4 == 0 and t_loop >= 8

    def body(edges_hbm, y_hbm, out_hbm, acc,
             i0, i1, i2, i3, r0, r1, zst,
             is0, is1, is2, is3, gs0, gs1, ss0, ss1):
        c = lax.axis_index("c")
        s = lax.axis_index("s")
        g = c * NS + s
        idxb = [i0, i1, i2, i3]
        rows = [r0, r1]
        isem = [is0, is1, is2, is3]
        gsem = [gs0, gs1]
        ssem = [ss0, ss1]
        zeros16 = jnp.zeros((16,), jnp.float32)

        def fill_zero(i, _):
            def fill_col(j, _):
                zst[i, pl.ds(j * 16, 16)] = zeros16
                return 0

            lax.fori_loop(0, d // 16, fill_col, 0)
            return 0

        lax.fori_loop(0, blk, fill_zero, 0)

        base = s * rpt

        def zero_blk(k, _):
            pltpu.sync_copy(zst, acc.at[pl.ds(base + k * blk, blk)])
            return 0

        lax.fori_loop(0, n_full, zero_blk, 0)
        if tail:
            pltpu.sync_copy(
                zst.at[pl.ds(0, tail)],
                acc.at[pl.ds(base + n_full * blk, tail)],
            )
        plsc.subcore_barrier()

        def wait_rows(sem_, buf):
            pltpu.make_async_copy(y_hbm.at[pl.ds(0, CH)], buf, sem_).wait()

        def step(j, bb, skip_a):
            b = bb % 2
            bn = (bb + 1) % 2
            if not skip_a:
                wait_rows(ssem[bn], rows[bn])
            pltpu.async_copy(
                edges_hbm.at[g, j + 3], idxb[(bb + 3) % 4], isem[(bb + 3) % 4]
            )
            pltpu.make_async_copy(
                edges_hbm.at[g, 0], idxb[(bb + 1) % 4], isem[(bb + 1) % 4]
            ).wait()
            pltpu.async_copy(
                y_hbm.at[idxb[(bb + 1) % 4].at[0]], rows[bn], gsem[bn]
            )
            wait_rows(gsem[b], rows[b])
            pltpu.async_copy(rows[b], acc.at[idxb[bb].at[1]], ssem[b], add=True)

        pltpu.async_copy(edges_hbm.at[g, 0], idxb[0], isem[0])
        pltpu.async_copy(edges_hbm.at[g, 1], idxb[1], isem[1])
        pltpu.async_copy(edges_hbm.at[g, 2], idxb[2], isem[2])
        pltpu.make_async_copy(edges_hbm.at[g, 0], idxb[0], isem[0]).wait()
        pltpu.async_copy(y_hbm.at[idxb[0].at[0]], rows[0], gsem[0])

        for bb in range(4):
            step(bb, bb, skip_a=(bb == 0))

        def group(gi, _):
            j0 = gi * 4
            for bb in range(4):
                step(j0 + bb, bb, skip_a=False)
            return 0

        lax.fori_loop(1, t_loop // 4, group, 0)

        wait_rows(ssem[(t_loop - 1) % 2], rows[(t_loop - 1) % 2])
        wait_rows(gsem[t_loop % 2], rows[t_loop % 2])
        pltpu.make_async_copy(
            edges_hbm.at[g, 0], idxb[(t_loop + 1) % 4], isem[(t_loop + 1) % 4]
        ).wait()
        pltpu.make_async_copy(
            edges_hbm.at[g, 0], idxb[(t_loop + 2) % 4], isem[(t_loop + 2) % 4]
        ).wait()
        plsc.subcore_barrier()

        def read_blk(k, _):
            pltpu.sync_copy(acc.at[pl.ds(base + k * blk, blk)], zst)
            pltpu.sync_copy(zst, out_hbm.at[c, pl.ds(base + k * blk, blk)])
            return 0

        lax.fori_loop(0, n_full, read_blk, 0)
        if tail:
            pltpu.sync_copy(
                acc.at[pl.ds(base + n_full * blk, tail)],
                zst.at[pl.ds(0, tail)],
            )
            pltpu.sync_copy(
                zst.at[pl.ds(0, tail)],
                out_hbm.at[c, pl.ds(base + n_full * blk, tail)],
            )

    return pl.kernel(
        body,
        out_type=jax.ShapeDtypeStruct((NC, n_pad, d), jnp.float32),
        mesh=_sc_mesh(),
        compiler_params=_NOTILE,
        scratch_types=[
            pltpu.VMEM_SHARED((n_pad, d), jnp.float32),
            pltpu.VMEM((2, CH), jnp.int32),
            pltpu.VMEM((2, CH), jnp.int32),
            pltpu.VMEM((2, CH), jnp.int32),
            pltpu.VMEM((2, CH), jnp.int32),
            pltpu.VMEM((CH, d), jnp.float32),
            pltpu.VMEM((CH, d), jnp.float32),
            pltpu.VMEM((blk, d), jnp.float32),
            pltpu.SemaphoreType.DMA,
            pltpu.SemaphoreType.DMA,
            pltpu.SemaphoreType.DMA,
            pltpu.SemaphoreType.DMA,
            pltpu.SemaphoreType.DMA,
            pltpu.SemaphoreType.DMA,
            pltpu.SemaphoreType.DMA,
            pltpu.SemaphoreType.DMA,
        ],
    )


def _tc_scale0(hist_ref, x_ref, dinv_ref, y1_ref):
    deg = 1.0 + hist_ref[0, :, 0:1] + hist_ref[1, :, 0:1]
    dinv = lax.rsqrt(deg)
    dinv_ref[...] = dinv
    y1_ref[...] = x_ref[...] * dinv


def _tc_scale1(part_ref, y1_ref, dinv_ref, y2_ref):
    dv = dinv_ref[...]
    y2_ref[...] = (part_ref[0] + part_ref[1] + y1_ref[...]) * (dv * dv)


def _tc_final(part_ref, y2_ref, dinv_ref, wt_ref, b_ref, o_ref):
    h2 = (part_ref[0] + part_ref[1] + y2_ref[...]) * dinv_ref[...]
    o_ref[...] = (
        jnp.dot(h2, wt_ref[...], preferred_element_type=jnp.float32) + b_ref[...]
    )


def kernel(x, edge_index, W, b):
    n, d = x.shape
    e = edge_index.shape[1]
    assert n % NS == 0 and d % 16 == 0

    t_loop = -(-e // (NW * CH))
    t_loop = -(-t_loop // 8) * 8
    n_groups = t_loop // 4
    e_pad = NW * t_loop * CH
    n_pad = -(-(n + 1) // (NS * 8)) * (NS * 8)

    src = edge_index[0]
    dst = edge_index[1]
    pad = e_pad - e
    if pad:
        src = jnp.concatenate([src, jnp.zeros((pad,), jnp.int32)])
        dst = jnp.concatenate([dst, jnp.full((pad,), n, jnp.int32)])
    dst_m = dst.reshape(NW, t_loop, CH)
    tg = NW * t_loop // 4
    per_core = tg // NS
    ga = max(4, int(round(SPLIT_A * per_core / 2.0)) * 2)
    gb = per_core - ga
    s4 = src.reshape(tg, 4, CH)
    d4 = dst.reshape(tg, 4, CH)
    inter = jnp.stack([s4, d4], axis=2).reshape(tg, 8, CH)
    padg = jnp.stack(
        [
            jnp.zeros((1, 4, CH), jnp.int32),
            jnp.full((1, 4, CH), n, jnp.int32),
        ],
        axis=2,
    ).reshape(1, 8, CH)
    edges8 = jnp.concatenate([inter, padg], axis=0)

    hist = _make_hist(n_pad, t_loop)(dst_m)
    hop = _make_hop_v3(n_pad, ga, gb, d)

    dinv, y1 = pl.pallas_call(
        _tc_scale0,
        out_shape=[
            jax.ShapeDtypeStruct((n, 1), jnp.float32),
            jax.ShapeDtypeStruct((n, d), jnp.float32),
        ],
    )(hist[:, :n, :], x)

    p = hop(edges8, y1)

    br = 2000 if n % 2000 == 0 else n
    grid = n // br
    y2 = pl.pallas_call(
        _tc_scale1,
        grid=(grid,),
        in_specs=[
            pl.BlockSpec((NC, br, d), lambda i: (0, i, 0)),
            pl.BlockSpec((br, d), lambda i: (i, 0)),
            pl.BlockSpec((br, 1), lambda i: (i, 0)),
        ],
        out_specs=pl.BlockSpec((br, d), lambda i: (i, 0)),
        out_shape=jax.ShapeDtypeStruct((n, d), jnp.float32),
    )(p[:, :n, :], y1, dinv)

    q = hop(edges8, y2)

    out = pl.pallas_call(
        _tc_final,
        grid=(grid,),
        in_specs=[
            pl.BlockSpec((NC, br, d), lambda i: (0, i, 0)),
            pl.BlockSpec((br, d), lambda i: (i, 0)),
            pl.BlockSpec((br, 1), lambda i: (i, 0)),
            pl.BlockSpec((d, d), lambda i: (0, 0)),
            pl.BlockSpec((1, d), lambda i: (0, 0)),
        ],
        out_specs=pl.BlockSpec((br, d), lambda i: (i, 0)),
        out_shape=jax.ShapeDtypeStruct((n, d), jnp.float32),
    )(q[:, :n, :], y2, dinv, W.T, b.reshape(1, d))
    return out


# speedup vs baseline: 1.4385x; 1.0933x over previous
import functools

import jax
import jax.numpy as jnp
from jax import lax
from jax.experimental import pallas as pl
from jax.experimental.pallas import tpu as pltpu
from jax.experimental.pallas import tpu_sc as plsc

NC = 2
SPLIT_A = 0.7
NS = 16
NW = NC * NS
CH = 128
HW = 16

_NOTILE = pltpu.CompilerParams(use_tc_tiling_on_sc=False)


def _sc_mesh():
    return plsc.VectorSubcoreMesh(
        core_axis_name="c", subcore_axis_name="s", num_cores=NC, num_subcores=NS
    )


def _make_hist(n_pad, ts):
    rpt = n_pad // NS

    def body(edges_hbm, out_hbm, acc, idx_v, ones_v, stage_v, gsem):
        c = lax.axis_index("c")
        s = lax.axis_index("s")
        g = c * NS + s
        ones16 = jnp.full((16,), 1.0, jnp.float32)
        zeros16 = jnp.zeros((16,), jnp.float32)

        def fill_ones(i, _):
            ones_v[i, :] = ones16
            return 0

        lax.fori_loop(0, CH, fill_ones, 0)

        def fill_zero(i, _):
            stage_v[i, :] = zeros16
            return 0

        lax.fori_loop(0, rpt, fill_zero, 0)

        base = s * rpt
        pltpu.sync_copy(stage_v, acc.at[pl.ds(base, rpt)])
        plsc.subcore_barrier()

        def chunk(j, _):
            pltpu.sync_copy(edges_hbm.at[g, j], idx_v)
            pltpu.sync_copy(ones_v, acc.at[idx_v], add=True)
            return 0

        lax.fori_loop(0, ts, chunk, 0)
        plsc.subcore_barrier()

        pltpu.sync_copy(acc.at[pl.ds(base, rpt)], stage_v)
        pltpu.sync_copy(stage_v, out_hbm.at[c, pl.ds(base, rpt)])

    return pl.kernel(
        body,
        out_type=jax.ShapeDtypeStruct((NC, n_pad, HW), jnp.float32),
        mesh=_sc_mesh(),
        compiler_params=_NOTILE,
        scratch_types=[
            pltpu.VMEM_SHARED((n_pad, HW), jnp.float32),
            pltpu.VMEM((CH,), jnp.int32),
            pltpu.VMEM((CH, HW), jnp.float32),
            pltpu.VMEM((rpt, HW), jnp.float32),
            pltpu.SemaphoreType.DMA,
        ],
    )


def _make_hop_serial(n_pad, t_loop, d):
    rpt = n_pad // NS
    blk = 104
    n_full = rpt // blk
    tail = rpt - n_full * blk
    assert tail % 8 == 0

    def body(edges_hbm, y_hbm, out_hbm, acc, idx_v, rows_v, zst, gsem):
        c = lax.axis_index("c")
        s = lax.axis_index("s")
        g = c * NS + s
        zeros16 = jnp.zeros((16,), jnp.float32)

        def fill_zero(i, _):
            def fill_col(j, _):
                zst[i, pl.ds(j * 16, 16)] = zeros16
                return 0

            lax.fori_loop(0, d // 16, fill_col, 0)
            return 0

        lax.fori_loop(0, blk, fill_zero, 0)

        base = s * rpt

        def zero_blk(k, _):
            pltpu.sync_copy(zst, acc.at[pl.ds(base + k * blk, blk)])
            return 0

        lax.fori_loop(0, n_full, zero_blk, 0)
        if tail:
            pltpu.sync_copy(
                zst.at[pl.ds(0, tail)],
                acc.at[pl.ds(base + n_full * blk, tail)],
            )
        plsc.subcore_barrier()

        def chunk(j, _):
            pltpu.sync_copy(edges_hbm.at[g, j], idx_v)
            pltpu.async_copy(y_hbm.at[idx_v.at[0]], rows_v, gsem).wait()
            pltpu.sync_copy(rows_v, acc.at[idx_v.at[1]], add=True)
            return 0

        lax.fori_loop(0, t_loop, chunk, 0)
        plsc.subcore_barrier()

        def read_blk(k, _):
            pltpu.sync_copy(acc.at[pl.ds(base + k * blk, blk)], zst)
            pltpu.sync_copy(zst, out_hbm.at[c, pl.ds(base + k * blk, blk)])
            return 0

        lax.fori_loop(0, n_full, read_blk, 0)
        if tail:
            pltpu.sync_copy(
                acc.at[pl.ds(base + n_full * blk, tail)],
                zst.at[pl.ds(0, tail)],
            )
            pltpu.sync_copy(
                zst.at[pl.ds(0, tail)],
                out_hbm.at[c, pl.ds(base + n_full * blk, tail)],
            )

    return pl.kernel(
        body,
        out_type=jax.ShapeDtypeStruct((NC, n_pad, d), jnp.float32),
        mesh=_sc_mesh(),
        compiler_params=_NOTILE,
        scratch_types=[
            pltpu.VMEM_SHARED((n_pad, d), jnp.float32),
            pltpu.VMEM((2, CH), jnp.int32),
            pltpu.VMEM((CH, d), jnp.float32),
            pltpu.VMEM((blk, d), jnp.float32),
            pltpu.SemaphoreType.DMA,
        ],
    )


def _make_hop_v3(n_pad, ga, gb, d):
    rpt = n_pad // NS
    blk = 80
    n_full = rpt // blk
    tail = rpt - n_full * blk
    assert tail % 8 == 0
    assert ga % 2 == 0 and gb % 2 == 0 and min(ga, gb) >= 4

    def body(edges_hbm, y_hbm, out_hbm, acc, ib0, ib1, r0, r1, zst,
             is0, is1, gs0, gs1, ss0, ss1):
        c = lax.axis_index("c")
        s = lax.axis_index("s")
        ng = jnp.where(c == 0, ga, gb)
        qbase = jnp.where(c == 0, s * ga, NS * ga + s * gb)
        idxb = [ib0, ib1]
        rows = [r0, r1]
        isem = [is0, is1]
        gsem = [gs0, gs1]
        ssem = [ss0, ss1]
        zeros16 = jnp.zeros((16,), jnp.float32)

        def fill_zero(i, _):
            def fill_col(j, _):
                zst[i, pl.ds(j * 16, 16)] = zeros16
                return 0

            lax.fori_loop(0, d // 16, fill_col, 0)
            return 0

        lax.fori_loop(0, blk, fill_zero, 0)

        base = s * rpt

        def zero_blk(k, _):
            pltpu.sync_copy(zst, acc.at[pl.ds(base + k * blk, blk)])
            return 0

        lax.fori_loop(0, n_full, zero_blk, 0)
        if tail:
            pltpu.sync_copy(
                zst.at[pl.ds(0, tail)],
                acc.at[pl.ds(base + n_full * blk, tail)],
            )
        plsc.subcore_barrier()

        def wait_rows(sem_, buf):
            pltpu.make_async_copy(y_hbm.at[pl.ds(0, CH)], buf, sem_).wait()

        def wait_idx(p):
            pltpu.make_async_copy(edges_hbm.at[qbase], idxb[p], isem[p]).wait()

        def start_idx(q, p):
            pltpu.async_copy(edges_hbm.at[qbase + q], idxb[p], isem[p])

        def start_gather(p, kk, b):
            pltpu.async_copy(
                y_hbm.at[idxb[p].at[2 * kk]], rows[b], gsem[b]
            )

        def start_scatter(p, kk, b):
            pltpu.async_copy(
                rows[b], acc.at[idxb[p].at[2 * kk + 1]], ssem[b], add=True
            )

        start_idx(0, 0)
        wait_idx(0)
        start_gather(0, 0, 0)
        start_idx(1, 1)
        start_gather(0, 1, 1)
        wait_rows(gsem[0], rows[0])
        start_scatter(0, 0, 0)
        for kk in (2, 3):
            b = kk % 2
            wait_rows(ssem[b], rows[b])
            start_gather(0, kk, b)
            wait_rows(gsem[1 - b], rows[1 - b])
            start_scatter(0, kk - 1, 1 - b)
        wait_idx(1)
        wait_rows(ssem[0], rows[0])
        start_gather(1, 0, 0)
        wait_rows(gsem[1], rows[1])
        start_scatter(0, 3, 1)
        wait_rows(ssem[1], rows[1])
        start_idx(2, 0)
        start_gather(1, 1, 1)
        wait_rows(gsem[0], rows[0])
        start_scatter(1, 0, 0)
        for kk in (2, 3):
            b = kk % 2
            wait_rows(ssem[b], rows[b])
            start_gather(1, kk, b)
            wait_rows(gsem[1 - b], rows[1 - b])
            start_scatter(1, kk - 1, 1 - b)

        def pair(i, _):
            qe = 2 + 2 * i
            for po in ((0, 1, qe), (1, 0, qe + 1)):
                p, pprev, q = po
                wait_idx(p)
                wait_rows(ssem[0], rows[0])
                start_gather(p, 0, 0)
                wait_rows(gsem[1], rows[1])
                start_scatter(pprev, 3, 1)
                wait_rows(ssem[1], rows[1])
                start_idx(q + 1, pprev)
                start_gather(p, 1, 1)
                wait_rows(gsem[0], rows[0])
                start_scatter(p, 0, 0)
                for kk in (2, 3):
                    b = kk % 2
                    wait_rows(ssem[b], rows[b])
                    start_gather(p, kk, b)
                    wait_rows(gsem[1 - b], rows[1 - b])
                    start_scatter(p, kk - 1, 1 - b)
            return 0

        lax.fori_loop(0, (ng - 2) // 2, pair, 0)

        wait_rows(gsem[1], rows[1])
        start_scatter(1, 3, 1)
        wait_rows(ssem[0], rows[0])
        wait_rows(ssem[1], rows[1])
        wait_idx(0)
        plsc.subcore_barrier()

        def read_blk(k, _):
            pltpu.sync_copy(acc.at[pl.ds(base + k * blk, blk)], zst)
            pltpu.sync_copy(zst, out_hbm.at[c, pl.ds(base + k * blk, blk)])
            return 0

        lax.fori_loop(0, n_full, read_blk, 0)
        if tail:
            pltpu.sync_copy(
                acc.at[pl.ds(base + n_full * blk, tail)],
                zst.at[pl.ds(0, tail)],
            )
            pltpu.sync_copy(
                zst.at[pl.ds(0, tail)],
                out_hbm.at[c, pl.ds(base + n_full * blk, tail)],
            )

    return pl.kernel(
        body,
        out_type=jax.ShapeDtypeStruct((NC, n_pad, d), jnp.float32),
        mesh=_sc_mesh(),
        scratch_types=[
            pltpu.VMEM_SHARED((n_pad, d), jnp.float32),
            pltpu.VMEM((8, CH), jnp.int32),
            pltpu.VMEM((8, CH), jnp.int32),
            pltpu.VMEM((CH, d), jnp.float32),
            pltpu.VMEM((CH, d), jnp.float32),
            pltpu.VMEM((blk, d), jnp.float32),
            pltpu.SemaphoreType.DMA,
            pltpu.SemaphoreType.DMA,
            pltpu.SemaphoreType.DMA,
            pltpu.SemaphoreType.DMA,
            pltpu.SemaphoreType.DMA,
            pltpu.SemaphoreType.DMA,
        ],
    )


def _make_hop(n_pad, t_loop, d):
    rpt = n_pad // NS
    blk = 104
    n_full = rpt // blk
    tail = rpt - n_full * blk
    assert tail % 8 == 0 and t_loop % 4 == 0 and t_loop >= 8

    def body(edges_hbm, y_hbm, out_hbm, acc,
             i0, i1, i2, i3, r0, r1, zst,
             is0, is1, is2, is3, gs0, gs1, ss0, ss1):
        c = lax.axis_index("c")
        s = lax.axis_index("s")
        g = c * NS + s
        idxb = [i0, i1, i2, i3]
        rows = [r0, r1]
        isem = [is0, is1, is2, is3]
        gsem = [gs0, gs1]
        ssem = [ss0, ss1]
        zeros16 = jnp.zeros((16,), jnp.float32)

        def fill_zero(i, _):
            def fill_col(j, _):
                zst[i, pl.ds(j * 16, 16)] = zeros16
                return 0

            lax.fori_loop(0, d // 16, fill_col, 0)
            return 0

        lax.fori_loop(0, blk, fill_zero, 0)

        base = s * rpt

        def zero_blk(k, _):
            pltpu.sync_copy(zst, acc.at[pl.ds(base + k * blk, blk)])
            return 0

        lax.fori_loop(0, n_full, zero_blk, 0)
        if tail:
            pltpu.sync_copy(
                zst.at[pl.ds(0, tail)],
                acc.at[pl.ds(base + n_full * blk, tail)],
            )
        plsc.subcore_barrier()

        def wait_rows(sem_, buf):
            pltpu.make_async_copy(y_hbm.at[pl.ds(0, CH)], buf, sem_).wait()

        def step(j, bb, skip_a):
            b = bb % 2
            bn = (bb + 1) % 2
            if not skip_a:
                wait_rows(ssem[bn], rows[bn])
            pltpu.async_copy(
                edges_hbm.at[g, j + 3], idxb[(bb + 3) % 4], isem[(bb + 3) % 4]
            )
            pltpu.make_async_copy(
                edges_hbm.at[g, 0], idxb[(bb + 1) % 4], isem[(bb + 1) % 4]
            ).wait()
            pltpu.async_copy(
                y_hbm.at[idxb[(bb + 1) % 4].at[0]], rows[bn], gsem[bn]
            )
            wait_rows(gsem[b], rows[b])
            pltpu.async_copy(rows[b], acc.at[idxb[bb].at[1]], ssem[b], add=True)

        pltpu.async_copy(edges_hbm.at[g, 0], idxb[0], isem[0])
        pltpu.async_copy(edges_hbm.at[g, 1], idxb[1], isem[1])
        pltpu.async_copy(edges_hbm.at[g, 2], idxb[2], isem[2])
        pltpu.make_async_copy(edges_hbm.at[g, 0], idxb[0], isem[0]).wait()
        pltpu.async_copy(y_hbm.at[idxb[0].at[0]], rows[0], gsem[0])

        for bb in range(4):
            step(bb, bb, skip_a=(bb == 0))

        def group(gi, _):
            j0 = gi * 4
            for bb in range(4):
                step(j0 + bb, bb, skip_a=False)
            return 0

        lax.fori_loop(1, t_loop // 4, group, 0)

        wait_rows(ssem[(t_loop - 1) % 2], rows[(t_loop - 1) % 2])
        wait_rows(gsem[t_loop % 2], rows[t_loop % 2])
        pltpu.make_async_copy(
            edges_hbm.at[g, 0], idxb[(t_loop + 1) % 4], isem[(t_loop + 1) % 4]
        ).wait()
        pltpu.make_async_copy(
            edges_hbm.at[g, 0], idxb[(t_loop + 2) % 4], isem[(t_loop + 2) % 4]
        ).wait()
        plsc.subcore_barrier()

        def read_blk(k, _):
            pltpu.sync_copy(acc.at[pl.ds(base + k * blk, blk)], zst)
            pltpu.sync_copy(zst, out_hbm.at[c, pl.ds(base + k * blk, blk)])
            return 0

        lax.fori_loop(0, n_full, read_blk, 0)
        if tail:
            pltpu.sync_copy(
                acc.at[pl.ds(base + n_full * blk, tail)],
                zst.at[pl.ds(0, tail)],
            )
            pltpu.sync_copy(
                zst.at[pl.ds(0, tail)],
                out_hbm.at[c, pl.ds(base + n_full * blk, tail)],
            )

    return pl.kernel(
        body,
        out_type=jax.ShapeDtypeStruct((NC, n_pad, d), jnp.float32),
        mesh=_sc_mesh(),
        compiler_params=_NOTILE,
        scratch_types=[
            pltpu.VMEM_SHARED((n_pad, d), jnp.float32),
            pltpu.VMEM((2, CH), jnp.int32),
            pltpu.VMEM((2, CH), jnp.int32),
            pltpu.VMEM((2, CH), jnp.int32),
            pltpu.VMEM((2, CH), jnp.int32),
            pltpu.VMEM((CH, d), jnp.float32),
            pltpu.VMEM((CH, d), jnp.float32),
            pltpu.VMEM((blk, d), jnp.float32),
            pltpu.SemaphoreType.DMA,
            pltpu.SemaphoreType.DMA,
            pltpu.SemaphoreType.DMA,
            pltpu.SemaphoreType.DMA,
            pltpu.SemaphoreType.DMA,
            pltpu.SemaphoreType.DMA,
            pltpu.SemaphoreType.DMA,
            pltpu.SemaphoreType.DMA,
        ],
    )


def _tc_scale0(hist_ref, x_ref, dinv_ref, y1_ref):
    deg = 1.0 + hist_ref[0, :, 0:1] + hist_ref[1, :, 0:1]
    dinv = lax.rsqrt(deg)
    dinv_ref[...] = dinv
    y1_ref[...] = x_ref[...] * dinv


def _tc_scale1(part_ref, y1_ref, dinv_ref, y2_ref):
    dv = dinv_ref[...]
    y2_ref[...] = (part_ref[0] + part_ref[1] + y1_ref[...]) * (dv * dv)


def _tc_final(part_ref, y2_ref, dinv_ref, wt_ref, b_ref, o_ref):
    h2 = (part_ref[0] + part_ref[1] + y2_ref[...]) * dinv_ref[...]
    o_ref[...] = (
        jnp.dot(h2, wt_ref[...], preferred_element_type=jnp.float32) + b_ref[...]
    )


def kernel(x, edge_index, W, b):
    n, d = x.shape
    e = edge_index.shape[1]
    assert n % NS == 0 and d % 16 == 0

    t_loop = -(-e // (NW * CH))
    t_loop = -(-t_loop // 8) * 8
    n_groups = t_loop // 4
    e_pad = NW * t_loop * CH
    n_pad = -(-(n + 1) // (NS * 8)) * (NS * 8)

    src = edge_index[0]
    dst = edge_index[1]
    pad = e_pad - e
    if pad:
        src = jnp.concatenate([src, jnp.zeros((pad,), jnp.int32)])
        dst = jnp.concatenate([dst, jnp.full((pad,), n, jnp.int32)])
    dst_m = dst.reshape(NW, t_loop, CH)
    tg = NW * t_loop // 4
    per_core = tg // NS
    ga = max(4, int(round(SPLIT_A * per_core / 2.0)) * 2)
    gb = per_core - ga
    s4 = src.reshape(tg, 4, CH)
    d4 = dst.reshape(tg, 4, CH)
    inter = jnp.stack([s4, d4], axis=2).reshape(tg, 8, CH)
    padg = jnp.stack(
        [
            jnp.zeros((1, 4, CH), jnp.int32),
            jnp.full((1, 4, CH), n, jnp.int32),
        ],
        axis=2,
    ).reshape(1, 8, CH)
    edges8 = jnp.concatenate([inter, padg], axis=0)

    hist = _make_hist(n_pad, t_loop)(dst_m)
    hop = _make_hop_v3(n_pad, ga, gb, d)

    dinv, y1 = pl.pallas_call(
        _tc_scale0,
        out_shape=[
            jax.ShapeDtypeStruct((n, 1), jnp.float32),
            jax.ShapeDtypeStruct((n, d), jnp.float32),
        ],
    )(hist[:, :n, :], x)

    p = hop(edges8, y1)

    br = 2000 if n % 2000 == 0 else n
    grid = n // br
    y2 = pl.pallas_call(
        _tc_scale1,
        grid=(grid,),
        in_specs=[
            pl.BlockSpec((NC, br, d), lambda i: (0, i, 0)),
            pl.BlockSpec((br, d), lambda i: (i, 0)),
            pl.BlockSpec((br, 1), lambda i: (i, 0)),
        ],
        out_specs=pl.BlockSpec((br, d), lambda i: (i, 0)),
        out_shape=jax.ShapeDtypeStruct((n, d), jnp.float32),
    )(p[:, :n, :], y1, dinv)

    q = hop(edges8, y2)

    out = pl.pallas_call(
        _tc_final,
        grid=(grid,),
        in_specs=[
            pl.BlockSpec((NC, br, d), lambda i: (0, i, 0)),
            pl.BlockSpec((br, d), lambda i: (i, 0)),
            pl.BlockSpec((br, 1), lambda i: (i, 0)),
            pl.BlockSpec((d, d), lambda i: (0, 0)),
            pl.BlockSpec((1, d), lambda i: (0, 0)),
        ],
        out_specs=pl.BlockSpec((br, d), lambda i: (i, 0)),
        out_shape=jax.ShapeDtypeStruct((n, d), jnp.float32),
    )(q[:, :n, :], y2, dinv, W.T, b.reshape(1, d))
    return out


# speedup vs baseline: 1.5789x; 1.0976x over previous
import functools

import jax
import jax.numpy as jnp
from jax import lax
from jax.experimental import pallas as pl
from jax.experimental.pallas import tpu as pltpu
from jax.experimental.pallas import tpu_sc as plsc

NC = 2
SPLIT_A = 0.9
NS = 16
NW = NC * NS
CH = 128
HW = 16

_NOTILE = pltpu.CompilerParams(use_tc_tiling_on_sc=False)


def _sc_mesh():
    return plsc.VectorSubcoreMesh(
        core_axis_name="c", subcore_axis_name="s", num_cores=NC, num_subcores=NS
    )


def _make_hist(n_pad, ts):
    rpt = n_pad // NS

    def body(edges_hbm, out_hbm, acc, idx_v, ones_v, stage_v, gsem):
        c = lax.axis_index("c")
        s = lax.axis_index("s")
        g = c * NS + s
        ones16 = jnp.full((16,), 1.0, jnp.float32)
        zeros16 = jnp.zeros((16,), jnp.float32)

        def fill_ones(i, _):
            ones_v[i, :] = ones16
            return 0

        lax.fori_loop(0, CH, fill_ones, 0)

        def fill_zero(i, _):
            stage_v[i, :] = zeros16
            return 0

        lax.fori_loop(0, rpt, fill_zero, 0)

        base = s * rpt
        pltpu.sync_copy(stage_v, acc.at[pl.ds(base, rpt)])
        plsc.subcore_barrier()

        def chunk(j, _):
            pltpu.sync_copy(edges_hbm.at[g, j], idx_v)
            pltpu.sync_copy(ones_v, acc.at[idx_v], add=True)
            return 0

        lax.fori_loop(0, ts, chunk, 0)
        plsc.subcore_barrier()

        pltpu.sync_copy(acc.at[pl.ds(base, rpt)], stage_v)
        pltpu.sync_copy(stage_v, out_hbm.at[c, pl.ds(base, rpt)])

    return pl.kernel(
        body,
        out_type=jax.ShapeDtypeStruct((NC, n_pad, HW), jnp.float32),
        mesh=_sc_mesh(),
        compiler_params=_NOTILE,
        scratch_types=[
            pltpu.VMEM_SHARED((n_pad, HW), jnp.float32),
            pltpu.VMEM((CH,), jnp.int32),
            pltpu.VMEM((CH, HW), jnp.float32),
            pltpu.VMEM((rpt, HW), jnp.float32),
            pltpu.SemaphoreType.DMA,
        ],
    )


def _make_hop_serial(n_pad, t_loop, d):
    rpt = n_pad // NS
    blk = 104
    n_full = rpt // blk
    tail = rpt - n_full * blk
    assert tail % 8 == 0

    def body(edges_hbm, y_hbm, out_hbm, acc, idx_v, rows_v, zst, gsem):
        c = lax.axis_index("c")
        s = lax.axis_index("s")
        g = c * NS + s
        zeros16 = jnp.zeros((16,), jnp.float32)

        def fill_zero(i, _):
            def fill_col(j, _):
                zst[i, pl.ds(j * 16, 16)] = zeros16
                return 0

            lax.fori_loop(0, d // 16, fill_col, 0)
            return 0

        lax.fori_loop(0, blk, fill_zero, 0)

        base = s * rpt

        def zero_blk(k, _):
            pltpu.sync_copy(zst, acc.at[pl.ds(base + k * blk, blk)])
            return 0

        lax.fori_loop(0, n_full, zero_blk, 0)
        if tail:
            pltpu.sync_copy(
                zst.at[pl.ds(0, tail)],
                acc.at[pl.ds(base + n_full * blk, tail)],
            )
        plsc.subcore_barrier()

        def chunk(j, _):
            pltpu.sync_copy(edges_hbm.at[g, j], idx_v)
            pltpu.async_copy(y_hbm.at[idx_v.at[0]], rows_v, gsem).wait()
            pltpu.sync_copy(rows_v, acc.at[idx_v.at[1]], add=True)
            return 0

        lax.fori_loop(0, t_loop, chunk, 0)
        plsc.subcore_barrier()

        def read_blk(k, _):
            pltpu.sync_copy(acc.at[pl.ds(base + k * blk, blk)], zst)
            pltpu.sync_copy(zst, out_hbm.at[c, pl.ds(base + k * blk, blk)])
            return 0

        lax.fori_loop(0, n_full, read_blk, 0)
        if tail:
            pltpu.sync_copy(
                acc.at[pl.ds(base + n_full * blk, tail)],
                zst.at[pl.ds(0, tail)],
            )
            pltpu.sync_copy(
                zst.at[pl.ds(0, tail)],
                out_hbm.at[c, pl.ds(base + n_full * blk, tail)],
            )

    return pl.kernel(
        body,
        out_type=jax.ShapeDtypeStruct((NC, n_pad, d), jnp.float32),
        mesh=_sc_mesh(),
        compiler_params=_NOTILE,
        scratch_types=[
            pltpu.VMEM_SHARED((n_pad, d), jnp.float32),
            pltpu.VMEM((2, CH), jnp.int32),
            pltpu.VMEM((CH, d), jnp.float32),
            pltpu.VMEM((blk, d), jnp.float32),
            pltpu.SemaphoreType.DMA,
        ],
    )


def _make_hop_v3(n_pad, ga, gb, d):
    rpt = n_pad // NS
    blk = 80
    n_full = rpt // blk
    tail = rpt - n_full * blk
    assert tail % 8 == 0
    assert ga % 2 == 0 and gb % 2 == 0 and min(ga, gb) >= 4

    def body(edges_hbm, y_hbm, out_hbm, acc, ib0, ib1, r0, r1, zst,
             is0, is1, gs0, gs1, ss0, ss1):
        c = lax.axis_index("c")
        s = lax.axis_index("s")
        ng = jnp.where(c == 0, ga, gb)
        qbase = jnp.where(c == 0, s * ga, NS * ga + s * gb)
        idxb = [ib0, ib1]
        rows = [r0, r1]
        isem = [is0, is1]
        gsem = [gs0, gs1]
        ssem = [ss0, ss1]
        zeros16 = jnp.zeros((16,), jnp.float32)

        def fill_zero(i, _):
            def fill_col(j, _):
                zst[i, pl.ds(j * 16, 16)] = zeros16
                return 0

            lax.fori_loop(0, d // 16, fill_col, 0)
            return 0

        lax.fori_loop(0, blk, fill_zero, 0)

        base = s * rpt

        def zero_blk(k, _):
            pltpu.sync_copy(zst, acc.at[pl.ds(base + k * blk, blk)])
            return 0

        lax.fori_loop(0, n_full, zero_blk, 0)
        if tail:
            pltpu.sync_copy(
                zst.at[pl.ds(0, tail)],
                acc.at[pl.ds(base + n_full * blk, tail)],
            )
        plsc.subcore_barrier()

        def wait_rows(sem_, buf):
            pltpu.make_async_copy(y_hbm.at[pl.ds(0, CH)], buf, sem_).wait()

        def wait_idx(p):
            pltpu.make_async_copy(edges_hbm.at[qbase], idxb[p], isem[p]).wait()

        def start_idx(q, p):
            pltpu.async_copy(edges_hbm.at[qbase + q], idxb[p], isem[p])

        def start_gather(p, kk, b):
            pltpu.async_copy(
                y_hbm.at[idxb[p].at[2 * kk]], rows[b], gsem[b]
            )

        def start_scatter(p, kk, b):
            pltpu.async_copy(
                rows[b], acc.at[idxb[p].at[2 * kk + 1]], ssem[b], add=True
            )

        start_idx(0, 0)
        wait_idx(0)
        start_gather(0, 0, 0)
        start_idx(1, 1)
        start_gather(0, 1, 1)
        wait_rows(gsem[0], rows[0])
        start_scatter(0, 0, 0)
        for kk in (2, 3):
            b = kk % 2
            wait_rows(ssem[b], rows[b])
            start_gather(0, kk, b)
            wait_rows(gsem[1 - b], rows[1 - b])
            start_scatter(0, kk - 1, 1 - b)
        wait_idx(1)
        wait_rows(ssem[0], rows[0])
        start_gather(1, 0, 0)
        wait_rows(gsem[1], rows[1])
        start_scatter(0, 3, 1)
        wait_rows(ssem[1], rows[1])
        start_idx(2, 0)
        start_gather(1, 1, 1)
        wait_rows(gsem[0], rows[0])
        start_scatter(1, 0, 0)
        for kk in (2, 3):
            b = kk % 2
            wait_rows(ssem[b], rows[b])
            start_gather(1, kk, b)
            wait_rows(gsem[1 - b], rows[1 - b])
            start_scatter(1, kk - 1, 1 - b)

        def pair(i, _):
            qe = 2 + 2 * i
            for po in ((0, 1, qe), (1, 0, qe + 1)):
                p, pprev, q = po
                wait_idx(p)
                wait_rows(ssem[0], rows[0])
                start_gather(p, 0, 0)
                wait_rows(gsem[1], rows[1])
                start_scatter(pprev, 3, 1)
                wait_rows(ssem[1], rows[1])
                start_idx(q + 1, pprev)
                start_gather(p, 1, 1)
                wait_rows(gsem[0], rows[0])
                start_scatter(p, 0, 0)
                for kk in (2, 3):
                    b = kk % 2
                    wait_rows(ssem[b], rows[b])
                    start_gather(p, kk, b)
                    wait_rows(gsem[1 - b], rows[1 - b])
                    start_scatter(p, kk - 1, 1 - b)
            return 0

        lax.fori_loop(0, (ng - 2) // 2, pair, 0)

        wait_rows(gsem[1], rows[1])
        start_scatter(1, 3, 1)
        wait_rows(ssem[0], rows[0])
        wait_rows(ssem[1], rows[1])
        wait_idx(0)
        plsc.subcore_barrier()

        def read_blk(k, _):
            pltpu.sync_copy(acc.at[pl.ds(base + k * blk, blk)], zst)
            pltpu.sync_copy(zst, out_hbm.at[c, pl.ds(base + k * blk, blk)])
            return 0

        lax.fori_loop(0, n_full, read_blk, 0)
        if tail:
            pltpu.sync_copy(
                acc.at[pl.ds(base + n_full * blk, tail)],
                zst.at[pl.ds(0, tail)],
            )
            pltpu.sync_copy(
                zst.at[pl.ds(0, tail)],
                out_hbm.at[c, pl.ds(base + n_full * blk, tail)],
            )

    return pl.kernel(
        body,
        out_type=jax.ShapeDtypeStruct((NC, n_pad, d), jnp.float32),
        mesh=_sc_mesh(),
        scratch_types=[
            pltpu.VMEM_SHARED((n_pad, d), jnp.float32),
            pltpu.VMEM((8, CH), jnp.int32),
            pltpu.VMEM((8, CH), jnp.int32),
            pltpu.VMEM((CH, d), jnp.float32),
            pltpu.VMEM((CH, d), jnp.float32),
            pltpu.VMEM((blk, d), jnp.float32),
            pltpu.SemaphoreType.DMA,
            pltpu.SemaphoreType.DMA,
            pltpu.SemaphoreType.DMA,
            pltpu.SemaphoreType.DMA,
            pltpu.SemaphoreType.DMA,
            pltpu.SemaphoreType.DMA,
        ],
    )


def _make_hop(n_pad, t_loop, d):
    rpt = n_pad // NS
    blk = 104
    n_full = rpt // blk
    tail = rpt - n_full * blk
    assert tail % 8 == 0 and t_loop % 4 == 0 and t_loop >= 8

    def body(edges_hbm, y_hbm, out_hbm, acc,
             i0, i1, i2, i3, r0, r1, zst,
             is0, is1, is2, is3, gs0, gs1, ss0, ss1):
        c = lax.axis_index("c")
        s = lax.axis_index("s")
        g = c * NS + s
        idxb = [i0, i1, i2, i3]
        rows = [r0, r1]
        isem = [is0, is1, is2, is3]
        gsem = [gs0, gs1]
        ssem = [ss0, ss1]
        zeros16 = jnp.zeros((16,), jnp.float32)

        def fill_zero(i, _):
            def fill_col(j, _):
                zst[i, pl.ds(j * 16, 16)] = zeros16
                return 0

            lax.fori_loop(0, d // 16, fill_col, 0)
            return 0

        lax.fori_loop(0, blk, fill_zero, 0)

        base = s * rpt

        def zero_blk(k, _):
            pltpu.sync_copy(zst, acc.at[pl.ds(base + k * blk, blk)])
            return 0

        lax.fori_loop(0, n_full, zero_blk, 0)
        if tail:
            pltpu.sync_copy(
                zst.at[pl.ds(0, tail)],
                acc.at[pl.ds(base + n_full * blk, tail)],
            )
        plsc.subcore_barrier()

        def wait_rows(sem_, buf):
            pltpu.make_async_copy(y_hbm.at[pl.ds(0, CH)], buf, sem_).wait()

        def step(j, bb, skip_a):
            b = bb % 2
            bn = (bb + 1) % 2
            if not skip_a:
                wait_rows(ssem[bn], rows[bn])
            pltpu.async_copy(
                edges_hbm.at[g, j + 3], idxb[(bb + 3) % 4], isem[(bb + 3) % 4]
            )
            pltpu.make_async_copy(
                edges_hbm.at[g, 0], idxb[(bb + 1) % 4], isem[(bb + 1) % 4]
            ).wait()
            pltpu.async_copy(
                y_hbm.at[idxb[(bb + 1) % 4].at[0]], rows[bn], gsem[bn]
            )
            wait_rows(gsem[b], rows[b])
            pltpu.async_copy(rows[b], acc.at[idxb[bb].at[1]], ssem[b], add=True)

        pltpu.async_copy(edges_hbm.at[g, 0], idxb[0], isem[0])
        pltpu.async_copy(edges_hbm.at[g, 1], idxb[1], isem[1])
        pltpu.async_copy(edges_hbm.at[g, 2], idxb[2], isem[2])
        pltpu.make_async_copy(edges_hbm.at[g, 0], idxb[0], isem[0]).wait()
        pltpu.async_copy(y_hbm.at[idxb[0].at[0]], rows[0], gsem[0])

        for bb in range(4):
            step(bb, bb, skip_a=(bb == 0))

        def group(gi, _):
            j0 = gi * 4
            for bb in range(4):
                step(j0 + bb, bb, skip_a=False)
            return 0

        lax.fori_loop(1, t_loop // 4, group, 0)

        wait_rows(ssem[(t_loop - 1) % 2], rows[(t_loop - 1) % 2])
        wait_rows(gsem[t_loop % 2], rows[t_loop % 2])
        pltpu.make_async_copy(
            edges_hbm.at[g, 0], idxb[(t_loop + 1) % 4], isem[(t_loop + 1) % 4]
        ).wait()
        pltpu.make_async_copy(
            edges_hbm.at[g, 0], idxb[(t_loop + 2) % 4], isem[(t_loop + 2) % 4]
        ).wait()
        plsc.subcore_barrier()

        def read_blk(k, _):
            pltpu.sync_copy(acc.at[pl.ds(base + k * blk, blk)], zst)
            pltpu.sync_copy(zst, out_hbm.at[c, pl.ds(base + k * blk, blk)])
            return 0

        lax.fori_loop(0, n_full, read_blk, 0)
        if tail:
            pltpu.sync_copy(
                acc.at[pl.ds(base + n_full * blk, tail)],
                zst.at[pl.ds(0, tail)],
            )
            pltpu.sync_copy(
                zst.at[pl.ds(0, tail)],
                out_hbm.at[c, pl.ds(base + n_full * blk, tail)],
            )

    return pl.kernel(
        body,
        out_type=jax.ShapeDtypeStruct((NC, n_pad, d), jnp.float32),
        mesh=_sc_mesh(),
        compiler_params=_NOTILE,
        scratch_types=[
            pltpu.VMEM_SHARED((n_pad, d), jnp.float32),
            pltpu.VMEM((2, CH), jnp.int32),
            pltpu.VMEM((2, CH), jnp.int32),
            pltpu.VMEM((2, CH), jnp.int32),
            pltpu.VMEM((2, CH), jnp.int32),
            pltpu.VMEM((CH, d), jnp.float32),
            pltpu.VMEM((CH, d), jnp.float32),
            pltpu.VMEM((blk, d), jnp.float32),
            pltpu.SemaphoreType.DMA,
            pltpu.SemaphoreType.DMA,
            pltpu.SemaphoreType.DMA,
            pltpu.SemaphoreType.DMA,
            pltpu.SemaphoreType.DMA,
            pltpu.SemaphoreType.DMA,
            pltpu.SemaphoreType.DMA,
            pltpu.SemaphoreType.DMA,
        ],
    )


def _tc_scale0(hist_ref, x_ref, dinv_ref, y1_ref):
    deg = 1.0 + hist_ref[0, :, 0:1] + hist_ref[1, :, 0:1]
    dinv = lax.rsqrt(deg)
    dinv_ref[...] = dinv
    y1_ref[...] = x_ref[...] * dinv


def _tc_scale1(part_ref, y1_ref, dinv_ref, y2_ref):
    dv = dinv_ref[...]
    y2_ref[...] = (part_ref[0] + part_ref[1] + y1_ref[...]) * (dv * dv)


def _tc_final(part_ref, y2_ref, dinv_ref, wt_ref, b_ref, o_ref):
    h2 = (part_ref[0] + part_ref[1] + y2_ref[...]) * dinv_ref[...]
    o_ref[...] = (
        jnp.dot(h2, wt_ref[...], preferred_element_type=jnp.float32) + b_ref[...]
    )


def kernel(x, edge_index, W, b):
    n, d = x.shape
    e = edge_index.shape[1]
    assert n % NS == 0 and d % 16 == 0

    t_loop = -(-e // (NW * CH))
    t_loop = -(-t_loop // 8) * 8
    n_groups = t_loop // 4
    e_pad = NW * t_loop * CH
    n_pad = -(-(n + 1) // (NS * 8)) * (NS * 8)

    src = edge_index[0]
    dst = edge_index[1]
    pad = e_pad - e
    if pad:
        src = jnp.concatenate([src, jnp.zeros((pad,), jnp.int32)])
        dst = jnp.concatenate([dst, jnp.full((pad,), n, jnp.int32)])
    dst_m = dst.reshape(NW, t_loop, CH)
    tg = NW * t_loop // 4
    per_core = tg // NS
    ga = max(4, int(round(SPLIT_A * per_core / 2.0)) * 2)
    gb = per_core - ga
    s4 = src.reshape(tg, 4, CH)
    d4 = dst.reshape(tg, 4, CH)
    inter = jnp.stack([s4, d4], axis=2).reshape(tg, 8, CH)
    padg = jnp.stack(
        [
            jnp.zeros((1, 4, CH), jnp.int32),
            jnp.full((1, 4, CH), n, jnp.int32),
        ],
        axis=2,
    ).reshape(1, 8, CH)
    edges8 = jnp.concatenate([inter, padg], axis=0)

    hist = _make_hist(n_pad, t_loop)(dst_m)
    hop = _make_hop_v3(n_pad, ga, gb, d)

    dinv, y1 = pl.pallas_call(
        _tc_scale0,
        out_shape=[
            jax.ShapeDtypeStruct((n, 1), jnp.float32),
            jax.ShapeDtypeStruct((n, d), jnp.float32),
        ],
    )(hist[:, :n, :], x)

    p = hop(edges8, y1)

    br = 2000 if n % 2000 == 0 else n
    grid = n // br
    y2 = pl.pallas_call(
        _tc_scale1,
        grid=(grid,),
        in_specs=[
            pl.BlockSpec((NC, br, d), lambda i: (0, i, 0)),
            pl.BlockSpec((br, d), lambda i: (i, 0)),
            pl.BlockSpec((br, 1), lambda i: (i, 0)),
        ],
        out_specs=pl.BlockSpec((br, d), lambda i: (i, 0)),
        out_shape=jax.ShapeDtypeStruct((n, d), jnp.float32),
    )(p[:, :n, :], y1, dinv)

    q = hop(edges8, y2)

    out = pl.pallas_call(
        _tc_final,
        grid=(grid,),
        in_specs=[
            pl.BlockSpec((NC, br, d), lambda i: (0, i, 0)),
            pl.BlockSpec((br, d), lambda i: (i, 0)),
            pl.BlockSpec((br, 1), lambda i: (i, 0)),
            pl.BlockSpec((d, d), lambda i: (0, 0)),
            pl.BlockSpec((1, d), lambda i: (0, 0)),
        ],
        out_specs=pl.BlockSpec((br, d), lambda i: (i, 0)),
        out_shape=jax.ShapeDtypeStruct((n, d), jnp.float32),
    )(q[:, :n, :], y2, dinv, W.T, b.reshape(1, d))
    return out
